# Initial kernel scaffold; baseline (speedup 1.0000x reference)
import jax
import jax.numpy as jnp
from jax.experimental import pallas as pl


def kernel(x, edge_index, batch, W0, b0, W1, b1, W2, b2, Wn, bn, Wf1, bf1, Wf2, bf2):
    raise NotImplementedError("write your pallas kernel here")


# speedup vs baseline: 5.4730x; 5.4730x over previous
import functools

import jax
import jax.numpy as jnp
from jax import lax
from jax.experimental import pallas as pl
from jax.experimental.pallas import tpu as pltpu
from jax.experimental.pallas import tpu_sc as plsc

N = 10000
NH = N // 2
NACC = NH + 8
D = 256
DH = 128
E = 160000
G = 64
H = 512

CH = 80
TILES = 16
EPT = E // TILES
ROWS_PER_TILE = EPT // CH

NCHUNK = 125
ZR = NH // NCHUNK

RB = 400
GRID = N // RB

EROWS = 1250
ECOLS = 128


def _mesh():
    return plsc.VectorSubcoreMesh(core_axis_name="c", subcore_axis_name="s",
                                  num_cores=2, num_subcores=16)


def _fill_vmem_rows(ref, nrows, ncols, value):
    v = jnp.full((16,), value, jnp.float32)

    def body(i, carry):
        for k in range(ncols // 16):
            ref[i, pl.ds(k * 16, 16)] = v
        return carry

    lax.fori_loop(0, nrows, body, 0)


def _per_tile_chunks(s, fn):
    for m in range((NCHUNK + TILES - 1) // TILES):
        k = s + TILES * m

        @pl.when(k < NCHUNK)
        def _(k=k):
            fn(k * ZR)


def _idx_prep_tc(dst2d):
    def body(d_ref, a_ref, b_ref):
        dv = d_ref[...]
        a_ref[...] = jnp.where(dv < NH, dv, NH)
        b_ref[...] = jnp.where(dv >= NH, dv - NH, NH)

    full = pl.BlockSpec((EROWS, ECOLS), lambda: (0, 0))
    return pl.pallas_call(
        body,
        grid=(),
        in_specs=[full],
        out_specs=[full, full],
        out_shape=[jax.ShapeDtypeStruct((EROWS, ECOLS), jnp.int32),
                   jax.ShapeDtypeStruct((EROWS, ECOLS), jnp.int32)],
    )(dst2d)


def _deg_sc(dstA, dstB):
    @functools.partial(
        pl.kernel,
        out_type=jax.ShapeDtypeStruct((N, DH), jnp.float32),
        mesh=_mesh(),
        scratch_types=[
            pltpu.VMEM((ROWS_PER_TILE, CH), jnp.int32),
            pltpu.VMEM((CH, DH), jnp.float32),
            pltpu.VMEM((ZR, DH), jnp.float32),
            pltpu.VMEM_SHARED((NACC, DH), jnp.float32),
        ],
    )
    def k(dstA_hbm, dstB_hbm, out_hbm, idx_v, ones_v, zbuf_v, acc_sh):
        c = lax.axis_index("c")
        s = lax.axis_index("s")

        _fill_vmem_rows(ones_v, CH, DH, 1.0)
        _fill_vmem_rows(zbuf_v, ZR, DH, 0.0)
        _per_tile_chunks(
            s, lambda b: pltpu.sync_copy(zbuf_v, acc_sh.at[pl.ds(b, ZR)]))

        @pl.when(c == 0)
        def _():
            pltpu.sync_copy(dstA_hbm.at[s], idx_v)

        @pl.when(c == 1)
        def _():
            pltpu.sync_copy(dstB_hbm.at[s], idx_v)

        plsc.subcore_barrier()

        def body(j, carry):
            pltpu.sync_copy(ones_v, acc_sh.at[idx_v.at[j]], add=True)
            return carry

        lax.fori_loop(0, ROWS_PER_TILE, body, 0)
        plsc.subcore_barrier()

        def drain(b, base):
            pltpu.sync_copy(acc_sh.at[pl.ds(b, ZR)],
                            out_hbm.at[pl.ds(base + b, ZR)])

        @pl.when(c == 0)
        def _():
            _per_tile_chunks(s, lambda b: drain(b, 0))

        @pl.when(c == 1)
        def _():
            _per_tile_chunks(s, lambda b: drain(b, NH))

    return k(dstA, dstB)


def _spmm_sc(x0, x1, src3d, dstA, dstB):
    @functools.partial(
        pl.kernel,
        out_type=[jax.ShapeDtypeStruct((N, DH), jnp.float32),
                  jax.ShapeDtypeStruct((N, DH), jnp.float32)],
        mesh=_mesh(),
        scratch_types=[
            pltpu.VMEM((ROWS_PER_TILE, CH), jnp.int32),
            pltpu.VMEM((ROWS_PER_TILE, CH), jnp.int32),
            pltpu.VMEM((ROWS_PER_TILE, CH), jnp.int32),
            pltpu.VMEM((CH, DH), jnp.float32),
            pltpu.VMEM((ZR, DH), jnp.float32),
            pltpu.VMEM_SHARED((NACC, DH), jnp.float32),
        ],
    )
    def k(x0_hbm, x1_hbm, src_hbm, dstA_hbm, dstB_hbm, y0_hbm, y1_hbm,
          src_v, dstA_v, dstB_v, rows_v, zbuf_v, acc_sh):
        c = lax.axis_index("c")
        s = lax.axis_index("s")

        _fill_vmem_rows(zbuf_v, ZR, DH, 0.0)
        pltpu.sync_copy(src_hbm.at[s], src_v)
        pltpu.sync_copy(dstA_hbm.at[s], dstA_v)
        pltpu.sync_copy(dstB_hbm.at[s], dstB_v)

        def one_pass(x_hbm, dst_v, y_hbm, out_base):
            _per_tile_chunks(
                s, lambda b: pltpu.sync_copy(zbuf_v, acc_sh.at[pl.ds(b, ZR)]))
            plsc.subcore_barrier()

            def body(j, carry):
                pltpu.sync_copy(x_hbm.at[src_v.at[j]], rows_v)
                pltpu.sync_copy(rows_v, acc_sh.at[dst_v.at[j]], add=True)
                return carry

            lax.fori_loop(0, ROWS_PER_TILE, body, 0)
            plsc.subcore_barrier()
            _per_tile_chunks(
                s, lambda b: pltpu.sync_copy(
                    acc_sh.at[pl.ds(b, ZR)],
                    y_hbm.at[pl.ds(out_base + b, ZR)]))

        def both_passes(x_hbm, y_hbm):
            one_pass(x_hbm, dstA_v, y_hbm, 0)
            plsc.subcore_barrier()
            one_pass(x_hbm, dstB_v, y_hbm, NH)

        @pl.when(c == 0)
        def _():
            both_passes(x0_hbm, y0_hbm)

        @pl.when(c == 1)
        def _():
            both_passes(x1_hbm, y1_hbm)

    return k(x0, x1, src3d, dstA, dstB)


def _dinv_of(deg_ref):
    return lax.rsqrt(deg_ref[:, 0:1] + 1.0)


def _half_shapes():
    return [jax.ShapeDtypeStruct((N, DH), jnp.float32) for _ in range(2)]


def _tc_first(x, w0, deg):
    def body(x_ref, w_ref, deg_ref, o0_ref, o1_ref):
        dinv = _dinv_of(deg_ref)
        xw = jnp.dot(x_ref[...], w_ref[...], preferred_element_type=jnp.float32)
        xt = dinv * xw
        o0_ref[...] = xt[:, :DH]
        o1_ref[...] = xt[:, DH:]

    half = pl.BlockSpec((RB, DH), lambda i: (i, 0))
    return pl.pallas_call(
        body,
        grid=(GRID,),
        in_specs=[
            pl.BlockSpec((RB, D), lambda i: (i, 0)),
            pl.BlockSpec((D, D), lambda i: (0, 0)),
            half,
        ],
        out_specs=[half, half],
        out_shape=_half_shapes(),
    )(x, w0, deg)


def _tc_mid(y0, y1, p0, p1, deg, b, w):
    def body(y0_ref, y1_ref, p0_ref, p1_ref, deg_ref, b_ref, w_ref,
             o0_ref, o1_ref):
        dinv = _dinv_of(deg_ref)
        h0 = jax.nn.relu(dinv * (y0_ref[...] + p0_ref[...]) + b_ref[0:1, :DH])
        h1 = jax.nn.relu(dinv * (y1_ref[...] + p1_ref[...]) + b_ref[0:1, DH:])
        h = jnp.concatenate([h0, h1], axis=1)
        xw = jnp.dot(h, w_ref[...], preferred_element_type=jnp.float32)
        xt = dinv * xw
        o0_ref[...] = xt[:, :DH]
        o1_ref[...] = xt[:, DH:]

    half = pl.BlockSpec((RB, DH), lambda i: (i, 0))
    return pl.pallas_call(
        body,
        grid=(GRID,),
        in_specs=[
            half, half, half, half, half,
            pl.BlockSpec((1, D), lambda i: (0, 0)),
            pl.BlockSpec((D, D), lambda i: (0, 0)),
        ],
        out_specs=[half, half],
        out_shape=_half_shapes(),
    )(y0, y1, p0, p1, deg, b, w)


def _tc_final(y0, y1, p0, p1, deg, b2, wnT, bn, batch3, wf1, bf1, wf2, bf2):
    def body(y0_ref, y1_ref, p0_ref, p1_ref, deg_ref, b_ref, wn_ref, bn_ref,
             batch_ref, wf1_ref, bf1_ref, wf2_ref, bf2_ref,
             node_ref, fea_ref, pooled_acc, cnt_acc):
        i = pl.program_id(0)

        @pl.when(i == 0)
        def _():
            pooled_acc[...] = jnp.zeros_like(pooled_acc)
            cnt_acc[...] = jnp.zeros_like(cnt_acc)

        dinv = _dinv_of(deg_ref)
        h0 = jax.nn.relu(dinv * (y0_ref[...] + p0_ref[...]) + b_ref[0:1, :DH])
        h1 = jax.nn.relu(dinv * (y1_ref[...] + p1_ref[...]) + b_ref[0:1, DH:])
        h = jnp.concatenate([h0, h1], axis=1)

        npv = jnp.sum(h * wn_ref[...], axis=1) + bn_ref[0, 0]
        node_ref[...] = jax.nn.sigmoid(npv).reshape(1, 1, RB)

        brow = batch_ref[0, 0, :]
        seg = lax.broadcasted_iota(jnp.int32, (G, RB), 0)
        onehot = jnp.where(seg == brow[None, :], 1.0, 0.0).astype(jnp.float32)
        pooled_acc[...] += jnp.dot(onehot, h,
                                   preferred_element_type=jnp.float32)
        cnt = jnp.sum(onehot, axis=1)
        cnt_acc[...] += jnp.broadcast_to(cnt[:, None], (G, 128))

        @pl.when(i == GRID - 1)
        def _():
            cntcol = cnt_acc[:, 0:1]
            pooled = pooled_acc[...] / jnp.maximum(cntcol, 1.0)
            f1 = jax.nn.relu(
                jnp.dot(pooled, wf1_ref[...],
                        preferred_element_type=jnp.float32) + bf1_ref[0:1, :])
            f2 = jnp.dot(f1, wf2_ref[...],
                         preferred_element_type=jnp.float32) + bf2_ref[0:1, :]
            fea_ref[...] = jax.nn.sigmoid(f2)

    half = pl.BlockSpec((RB, DH), lambda i: (i, 0))
    return pl.pallas_call(
        body,
        grid=(GRID,),
        in_specs=[
            half, half, half, half, half,
            pl.BlockSpec((1, D), lambda i: (0, 0)),
            pl.BlockSpec((1, D), lambda i: (0, 0)),
            pl.BlockSpec((1, 1), lambda i: (0, 0)),
            pl.BlockSpec((1, 1, RB), lambda i: (i, 0, 0)),
            pl.BlockSpec((D, H), lambda i: (0, 0)),
            pl.BlockSpec((1, H), lambda i: (0, 0)),
            pl.BlockSpec((H, D), lambda i: (0, 0)),
            pl.BlockSpec((1, D), lambda i: (0, 0)),
        ],
        out_specs=[
            pl.BlockSpec((1, 1, RB), lambda i: (i, 0, 0)),
            pl.BlockSpec((G, D), lambda i: (0, 0)),
        ],
        out_shape=[jax.ShapeDtypeStruct((GRID, 1, RB), jnp.float32),
                   jax.ShapeDtypeStruct((G, D), jnp.float32)],
        scratch_shapes=[
            pltpu.VMEM((G, D), jnp.float32),
            pltpu.VMEM((G, 128), jnp.float32),
        ],
    )(y0, y1, p0, p1, deg, b2, wnT, bn, batch3, wf1, bf1, wf2, bf2)


def kernel(x, edge_index, batch, W0, b0, W1, b1, W2, b2, Wn, bn,
           Wf1, bf1, Wf2, bf2):
    src3d = edge_index[0].reshape(TILES, ROWS_PER_TILE, CH)
    dstA2, dstB2 = _idx_prep_tc(edge_index[1].reshape(EROWS, ECOLS))
    dstA = dstA2.reshape(TILES, ROWS_PER_TILE, CH)
    dstB = dstB2.reshape(TILES, ROWS_PER_TILE, CH)

    deg = _deg_sc(dstA, dstB)

    p0, p1 = _tc_first(x, W0, deg)
    y0, y1 = _spmm_sc(p0, p1, src3d, dstA, dstB)
    p0, p1 = _tc_mid(y0, y1, p0, p1, deg, b0.reshape(1, D), W1)
    y0, y1 = _spmm_sc(p0, p1, src3d, dstA, dstB)
    p0, p1 = _tc_mid(y0, y1, p0, p1, deg, b1.reshape(1, D), W2)
    y0, y1 = _spmm_sc(p0, p1, src3d, dstA, dstB)

    node3, fea = _tc_final(
        y0, y1, p0, p1, deg, b2.reshape(1, D), Wn.reshape(1, D),
        bn.reshape(1, 1), batch.reshape(GRID, 1, RB),
        Wf1, bf1.reshape(1, H), Wf2, bf2.reshape(1, D))

    return node3.reshape(N), fea


# speedup vs baseline: 7.7566x; 1.4172x over previous
import functools

import jax
import jax.numpy as jnp
from jax import lax
from jax.experimental import pallas as pl
from jax.experimental.pallas import tpu as pltpu
from jax.experimental.pallas import tpu_sc as plsc

N = 10000
NH = N // 2
NACC = NH + 8
D = 256
DH = 128
E = 160000
G = 64
H = 512

CH = 80
TILES = 16
EPT = E // TILES
ROWS_PER_TILE = EPT // CH

NCHUNK = 125
ZR = NH // NCHUNK

RB = 400
GRID = N // RB

EROWS = 1250
ECOLS = 128


def _mesh():
    return plsc.VectorSubcoreMesh(core_axis_name="c", subcore_axis_name="s",
                                  num_cores=2, num_subcores=16)


def _fill_vmem_rows(ref, nrows, ncols, value):
    v = jnp.full((16,), value, jnp.float32)

    def body(i, carry):
        for k in range(ncols // 16):
            ref[i, pl.ds(k * 16, 16)] = v
        return carry

    lax.fori_loop(0, nrows, body, 0)


def _per_tile_chunks(s, fn):
    for m in range((NCHUNK + TILES - 1) // TILES):
        k = s + TILES * m

        @pl.when(k < NCHUNK)
        def _(k=k):
            fn(k * ZR)


def _idx_prep_tc(dst2d):
    def body(d_ref, a_ref, b_ref):
        dv = d_ref[...]
        a_ref[...] = jnp.where(dv < NH, dv, NH)
        b_ref[...] = jnp.where(dv >= NH, dv - NH, NH)

    full = pl.BlockSpec((EROWS, ECOLS), lambda: (0, 0))
    return pl.pallas_call(
        body,
        grid=(),
        in_specs=[full],
        out_specs=[full, full],
        out_shape=[jax.ShapeDtypeStruct((EROWS, ECOLS), jnp.int32),
                   jax.ShapeDtypeStruct((EROWS, ECOLS), jnp.int32)],
    )(dst2d)


def _deg_sc(dstA, dstB):
    @functools.partial(
        pl.kernel,
        out_type=jax.ShapeDtypeStruct((N, DH), jnp.float32),
        mesh=_mesh(),
        scratch_types=[
            pltpu.VMEM((ROWS_PER_TILE, CH), jnp.int32),
            pltpu.VMEM((CH, DH), jnp.float32),
            pltpu.VMEM((ZR, DH), jnp.float32),
            pltpu.VMEM_SHARED((NACC, DH), jnp.float32),
        ],
    )
    def k(dstA_hbm, dstB_hbm, out_hbm, idx_v, ones_v, zbuf_v, acc_sh):
        c = lax.axis_index("c")
        s = lax.axis_index("s")

        _fill_vmem_rows(ones_v, CH, DH, 1.0)
        _fill_vmem_rows(zbuf_v, ZR, DH, 0.0)
        _per_tile_chunks(
            s, lambda b: pltpu.sync_copy(zbuf_v, acc_sh.at[pl.ds(b, ZR)]))

        @pl.when(c == 0)
        def _():
            pltpu.sync_copy(dstA_hbm.at[s], idx_v)

        @pl.when(c == 1)
        def _():
            pltpu.sync_copy(dstB_hbm.at[s], idx_v)

        plsc.subcore_barrier()

        def body(j, carry):
            pltpu.sync_copy(ones_v, acc_sh.at[idx_v.at[j]], add=True)
            return carry

        lax.fori_loop(0, ROWS_PER_TILE, body, 0)
        plsc.subcore_barrier()

        def drain(b, base):
            pltpu.sync_copy(acc_sh.at[pl.ds(b, ZR)],
                            out_hbm.at[pl.ds(base + b, ZR)])

        @pl.when(c == 0)
        def _():
            _per_tile_chunks(s, lambda b: drain(b, 0))

        @pl.when(c == 1)
        def _():
            _per_tile_chunks(s, lambda b: drain(b, NH))

    return k(dstA, dstB)


def _spmm_sc(x0, x1, src3d, dstA, dstB):
    @functools.partial(
        pl.kernel,
        out_type=[jax.ShapeDtypeStruct((N, DH), jnp.float32),
                  jax.ShapeDtypeStruct((N, DH), jnp.float32)],
        mesh=_mesh(),
        scratch_types=[
            pltpu.VMEM((ROWS_PER_TILE, CH), jnp.int32),
            pltpu.VMEM((ROWS_PER_TILE, CH), jnp.int32),
            pltpu.VMEM((ROWS_PER_TILE, CH), jnp.int32),
            pltpu.VMEM((CH, DH), jnp.float32),
            pltpu.VMEM((CH, DH), jnp.float32),
            pltpu.VMEM((ZR, DH), jnp.float32),
            pltpu.VMEM_SHARED((NACC, DH), jnp.float32),
            pltpu.SemaphoreType.DMA,
            pltpu.SemaphoreType.DMA,
        ],
    )
    def k(x0_hbm, x1_hbm, src_hbm, dstA_hbm, dstB_hbm, y0_hbm, y1_hbm,
          src_v, dstA_v, dstB_v, rb0, rb1, zbuf_v, acc_sh, gsem0, gsem1):
        c = lax.axis_index("c")
        s = lax.axis_index("s")

        _fill_vmem_rows(zbuf_v, ZR, DH, 0.0)
        pltpu.sync_copy(src_hbm.at[s], src_v)
        pltpu.sync_copy(dstA_hbm.at[s], dstA_v)
        pltpu.sync_copy(dstB_hbm.at[s], dstB_v)

        def one_pass(x_hbm, dst_v, y_hbm, out_base):
            _per_tile_chunks(
                s, lambda b: pltpu.sync_copy(zbuf_v, acc_sh.at[pl.ds(b, ZR)]))
            plsc.subcore_barrier()

            pltpu.async_copy(x_hbm.at[src_v.at[0]], rb0, gsem0)

            def body(t, carry):
                j0 = 2 * t
                pltpu.async_copy(x_hbm.at[src_v.at[j0 + 1]], rb1, gsem1)
                pltpu.make_async_copy(x_hbm.at[src_v.at[j0]], rb0, gsem0
                                      ).wait()
                pltpu.sync_copy(rb0, acc_sh.at[dst_v.at[j0]], add=True)
                pltpu.async_copy(x_hbm.at[src_v.at[j0 + 2]], rb0, gsem0)
                pltpu.make_async_copy(x_hbm.at[src_v.at[j0 + 1]], rb1, gsem1
                                      ).wait()
                pltpu.sync_copy(rb1, acc_sh.at[dst_v.at[j0 + 1]], add=True)
                return carry

            lax.fori_loop(0, (ROWS_PER_TILE - 1) // 2, body, 0)
            pltpu.make_async_copy(
                x_hbm.at[src_v.at[ROWS_PER_TILE - 1]], rb0, gsem0).wait()
            pltpu.sync_copy(rb0, acc_sh.at[dst_v.at[ROWS_PER_TILE - 1]],
                            add=True)
            plsc.subcore_barrier()
            _per_tile_chunks(
                s, lambda b: pltpu.sync_copy(
                    acc_sh.at[pl.ds(b, ZR)],
                    y_hbm.at[pl.ds(out_base + b, ZR)]))

        def both_passes(x_hbm, y_hbm):
            one_pass(x_hbm, dstA_v, y_hbm, 0)
            plsc.subcore_barrier()
            one_pass(x_hbm, dstB_v, y_hbm, NH)

        @pl.when(c == 0)
        def _():
            both_passes(x0_hbm, y0_hbm)

        @pl.when(c == 1)
        def _():
            both_passes(x1_hbm, y1_hbm)

    return k(x0, x1, src3d, dstA, dstB)


def _dinv_of(deg_ref):
    return lax.rsqrt(deg_ref[:, 0:1] + 1.0)


def _half_shapes():
    return [jax.ShapeDtypeStruct((N, DH), jnp.float32) for _ in range(2)]


def _tc_first(x, w0, deg):
    def body(x_ref, w_ref, deg_ref, o0_ref, o1_ref):
        dinv = _dinv_of(deg_ref)
        xw = jnp.dot(x_ref[...], w_ref[...], preferred_element_type=jnp.float32)
        xt = dinv * xw
        o0_ref[...] = xt[:, :DH]
        o1_ref[...] = xt[:, DH:]

    half = pl.BlockSpec((RB, DH), lambda i: (i, 0))
    return pl.pallas_call(
        body,
        grid=(GRID,),
        in_specs=[
            pl.BlockSpec((RB, D), lambda i: (i, 0)),
            pl.BlockSpec((D, D), lambda i: (0, 0)),
            half,
        ],
        out_specs=[half, half],
        out_shape=_half_shapes(),
    )(x, w0, deg)


def _tc_mid(y0, y1, p0, p1, deg, b, w):
    def body(y0_ref, y1_ref, p0_ref, p1_ref, deg_ref, b_ref, w_ref,
             o0_ref, o1_ref):
        dinv = _dinv_of(deg_ref)
        h0 = jax.nn.relu(dinv * (y0_ref[...] + p0_ref[...]) + b_ref[0:1, :DH])
        h1 = jax.nn.relu(dinv * (y1_ref[...] + p1_ref[...]) + b_ref[0:1, DH:])
        h = jnp.concatenate([h0, h1], axis=1)
        xw = jnp.dot(h, w_ref[...], preferred_element_type=jnp.float32)
        xt = dinv * xw
        o0_ref[...] = xt[:, :DH]
        o1_ref[...] = xt[:, DH:]

    half = pl.BlockSpec((RB, DH), lambda i: (i, 0))
    return pl.pallas_call(
        body,
        grid=(GRID,),
        in_specs=[
            half, half, half, half, half,
            pl.BlockSpec((1, D), lambda i: (0, 0)),
            pl.BlockSpec((D, D), lambda i: (0, 0)),
        ],
        out_specs=[half, half],
        out_shape=_half_shapes(),
    )(y0, y1, p0, p1, deg, b, w)


def _tc_final(y0, y1, p0, p1, deg, b2, wnT, bn, batch3, wf1, bf1, wf2, bf2):
    def body(y0_ref, y1_ref, p0_ref, p1_ref, deg_ref, b_ref, wn_ref, bn_ref,
             batch_ref, wf1_ref, bf1_ref, wf2_ref, bf2_ref,
             node_ref, fea_ref, pooled_acc, cnt_acc):
        i = pl.program_id(0)

        @pl.when(i == 0)
        def _():
            pooled_acc[...] = jnp.zeros_like(pooled_acc)
            cnt_acc[...] = jnp.zeros_like(cnt_acc)

        dinv = _dinv_of(deg_ref)
        h0 = jax.nn.relu(dinv * (y0_ref[...] + p0_ref[...]) + b_ref[0:1, :DH])
        h1 = jax.nn.relu(dinv * (y1_ref[...] + p1_ref[...]) + b_ref[0:1, DH:])
        h = jnp.concatenate([h0, h1], axis=1)

        npv = jnp.sum(h * wn_ref[...], axis=1) + bn_ref[0, 0]
        node_ref[...] = jax.nn.sigmoid(npv).reshape(1, 1, RB)

        brow = batch_ref[0, 0, :]
        seg = lax.broadcasted_iota(jnp.int32, (G, RB), 0)
        onehot = jnp.where(seg == brow[None, :], 1.0, 0.0).astype(jnp.float32)
        pooled_acc[...] += jnp.dot(onehot, h,
                                   preferred_element_type=jnp.float32)
        cnt = jnp.sum(onehot, axis=1)
        cnt_acc[...] += jnp.broadcast_to(cnt[:, None], (G, 128))

        @pl.when(i == GRID - 1)
        def _():
            cntcol = cnt_acc[:, 0:1]
            pooled = pooled_acc[...] / jnp.maximum(cntcol, 1.0)
            f1 = jax.nn.relu(
                jnp.dot(pooled, wf1_ref[...],
                        preferred_element_type=jnp.float32) + bf1_ref[0:1, :])
            f2 = jnp.dot(f1, wf2_ref[...],
                         preferred_element_type=jnp.float32) + bf2_ref[0:1, :]
            fea_ref[...] = jax.nn.sigmoid(f2)

    half = pl.BlockSpec((RB, DH), lambda i: (i, 0))
    return pl.pallas_call(
        body,
        grid=(GRID,),
        in_specs=[
            half, half, half, half, half,
            pl.BlockSpec((1, D), lambda i: (0, 0)),
            pl.BlockSpec((1, D), lambda i: (0, 0)),
            pl.BlockSpec((1, 1), lambda i: (0, 0)),
            pl.BlockSpec((1, 1, RB), lambda i: (i, 0, 0)),
            pl.BlockSpec((D, H), lambda i: (0, 0)),
            pl.BlockSpec((1, H), lambda i: (0, 0)),
            pl.BlockSpec((H, D), lambda i: (0, 0)),
            pl.BlockSpec((1, D), lambda i: (0, 0)),
        ],
        out_specs=[
            pl.BlockSpec((1, 1, RB), lambda i: (i, 0, 0)),
            pl.BlockSpec((G, D), lambda i: (0, 0)),
        ],
        out_shape=[jax.ShapeDtypeStruct((GRID, 1, RB), jnp.float32),
                   jax.ShapeDtypeStruct((G, D), jnp.float32)],
        scratch_shapes=[
            pltpu.VMEM((G, D), jnp.float32),
            pltpu.VMEM((G, 128), jnp.float32),
        ],
    )(y0, y1, p0, p1, deg, b2, wnT, bn, batch3, wf1, bf1, wf2, bf2)


def kernel(x, edge_index, batch, W0, b0, W1, b1, W2, b2, Wn, bn,
           Wf1, bf1, Wf2, bf2):
    src3d = edge_index[0].reshape(TILES, ROWS_PER_TILE, CH)
    dstA2, dstB2 = _idx_prep_tc(edge_index[1].reshape(EROWS, ECOLS))
    dstA = dstA2.reshape(TILES, ROWS_PER_TILE, CH)
    dstB = dstB2.reshape(TILES, ROWS_PER_TILE, CH)

    deg = _deg_sc(dstA, dstB)

    p0, p1 = _tc_first(x, W0, deg)
    y0, y1 = _spmm_sc(p0, p1, src3d, dstA, dstB)
    p0, p1 = _tc_mid(y0, y1, p0, p1, deg, b0.reshape(1, D), W1)
    y0, y1 = _spmm_sc(p0, p1, src3d, dstA, dstB)
    p0, p1 = _tc_mid(y0, y1, p0, p1, deg, b1.reshape(1, D), W2)
    y0, y1 = _spmm_sc(p0, p1, src3d, dstA, dstB)

    node3, fea = _tc_final(
        y0, y1, p0, p1, deg, b2.reshape(1, D), Wn.reshape(1, D),
        bn.reshape(1, 1), batch.reshape(GRID, 1, RB),
        Wf1, bf1.reshape(1, H), Wf2, bf2.reshape(1, D))

    return node3.reshape(N), fea


# speedup vs baseline: 10.4980x; 1.3534x over previous
import functools

import jax
import jax.numpy as jnp
from jax import lax
from jax.experimental import pallas as pl
from jax.experimental.pallas import tpu as pltpu
from jax.experimental.pallas import tpu_sc as plsc

N = 10000
NH = N // 2
NACC = NH + 8
D = 256
DH = 128
E = 160000
G = 64
H = 512

CH = 80
TILES = 16
EPT = E // TILES
ROWS_PER_TILE = EPT // CH

NCHUNK = 125
ZR = NH // NCHUNK

RB = 400
GRID = N // RB

EROWS = 1250
ECOLS = 128


def _mesh():
    return plsc.VectorSubcoreMesh(core_axis_name="c", subcore_axis_name="s",
                                  num_cores=2, num_subcores=16)


def _fill_vmem_rows(ref, nrows, ncols, value):
    v = jnp.full((16,), value, jnp.float32)

    def body(i, carry):
        for k in range(ncols // 16):
            ref[i, pl.ds(k * 16, 16)] = v
        return carry

    lax.fori_loop(0, nrows, body, 0)


def _per_tile_chunks(s, fn):
    for m in range((NCHUNK + TILES - 1) // TILES):
        k = s + TILES * m

        @pl.when(k < NCHUNK)
        def _(k=k):
            fn(k * ZR)


LROWS = 128


def _part_sc(src3d, dst3d):
    list_ty = jax.ShapeDtypeStruct((TILES, LROWS, CH), jnp.int32)
    cnt_ty = jax.ShapeDtypeStruct((TILES * 16,), jnp.int32)

    @functools.partial(
        pl.kernel,
        out_type=[list_ty, list_ty, list_ty, list_ty, cnt_ty, cnt_ty],
        mesh=_mesh(),
        compiler_params=pltpu.CompilerParams(needs_layout_passes=False),
        scratch_types=[
            pltpu.VMEM((ROWS_PER_TILE, CH), jnp.int32),
            pltpu.VMEM((ROWS_PER_TILE, CH), jnp.int32),
            pltpu.VMEM((LROWS, CH), jnp.int32),
            pltpu.VMEM((LROWS, CH), jnp.int32),
            pltpu.VMEM((LROWS, CH), jnp.int32),
            pltpu.VMEM((LROWS, CH), jnp.int32),
            pltpu.VMEM((16,), jnp.int32),
        ],
    )
    def k(src_hbm, dst_hbm, srcA_hbm, dstA_hbm, srcB_hbm, dstB_hbm,
          nchA_hbm, nchB_hbm, src_v, dst_v, bAs, bAd, bBs, bBd, cnt_v):
        c = lax.axis_index("c")
        s = lax.axis_index("s")

        @pl.when(c == 0)
        def _():
            pltpu.sync_copy(src_hbm.at[s], src_v)
            pltpu.sync_copy(dst_hbm.at[s], dst_v)
            iota = lax.iota(jnp.int32, 16)

            def bc(x):
                return lax.broadcast_in_dim(x, (16,), ())

            def row(j, off):
                offA, offB = off
                for kk in range(CH // 16):
                    sv = src_v[j, pl.ds(kk * 16, 16)]
                    dv = dst_v[j, pl.ds(kk * 16, 16)]
                    mA = dv < NH
                    mB = dv >= NH
                    cumA = plsc.cumsum(jnp.where(mA, 1, 0))
                    posA = bc(offA) + cumA - 1
                    posB = bc(offB) + iota - cumA
                    plsc.store_scatter(bAs, [posA // CH, posA % CH], sv,
                                       mask=mA)
                    plsc.store_scatter(bAd, [posA // CH, posA % CH], dv,
                                       mask=mA)
                    plsc.store_scatter(bBs, [posB // CH, posB % CH], sv,
                                       mask=mB)
                    plsc.store_scatter(bBd, [posB // CH, posB % CH],
                                       dv - NH, mask=mB)
                    nA = jnp.sum(jnp.where(mA, 1, 0))
                    offA = offA + nA
                    offB = offB + 16 - nA
                return (offA, offB)

            offA, offB = lax.fori_loop(0, ROWS_PER_TILE, row,
                                       (jnp.int32(0), jnp.int32(0)))

            zsrc = jnp.zeros((16,), jnp.int32)
            zdst = jnp.full((16,), NH, jnp.int32)
            mall = iota < 16
            for t in range(CH // 16):
                pA = bc(offA + t * 16) + iota
                plsc.store_scatter(bAs, [pA // CH, pA % CH], zsrc, mask=mall)
                plsc.store_scatter(bAd, [pA // CH, pA % CH], zdst, mask=mall)
                pB = bc(offB + t * 16) + iota
                plsc.store_scatter(bBs, [pB // CH, pB % CH], zsrc, mask=mall)
                plsc.store_scatter(bBd, [pB // CH, pB % CH], zdst, mask=mall)

            pltpu.sync_copy(bAs, srcA_hbm.at[s])
            pltpu.sync_copy(bAd, dstA_hbm.at[s])
            pltpu.sync_copy(bBs, srcB_hbm.at[s])
            pltpu.sync_copy(bBd, dstB_hbm.at[s])

            nchunksA = (offA + CH - 1) // CH
            cnt_v[...] = jnp.where(iota == 0, bc(nchunksA), 0)
            pltpu.sync_copy(cnt_v, nchA_hbm.at[pl.ds(s * 16, 16)])
            nchunksB = (offB + CH - 1) // CH
            cnt_v[...] = jnp.where(iota == 0, bc(nchunksB), 0)
            pltpu.sync_copy(cnt_v, nchB_hbm.at[pl.ds(s * 16, 16)])

    return k(src3d, dst3d)


def _my_count(nch_v, s):
    vec = nch_v[pl.ds(s * 16, 16)]
    iota = lax.iota(jnp.int32, 16)
    return jnp.sum(jnp.where(iota == 0, vec, 0))


def _deg_sc(dstA, dstB, nchA, nchB):
    @functools.partial(
        pl.kernel,
        out_type=jax.ShapeDtypeStruct((N, DH), jnp.float32),
        mesh=_mesh(),
        compiler_params=pltpu.CompilerParams(needs_layout_passes=False),
        scratch_types=[
            pltpu.VMEM((LROWS, CH), jnp.int32),
            pltpu.VMEM((16 * TILES,), jnp.int32),
            pltpu.VMEM((16 * TILES,), jnp.int32),
            pltpu.VMEM((CH, DH), jnp.float32),
            pltpu.VMEM((ZR, DH), jnp.float32),
            pltpu.VMEM_SHARED((NACC, DH), jnp.float32),
        ],
    )
    def k(dstA_hbm, dstB_hbm, nchA_hbm, nchB_hbm, out_hbm,
          idx_v, nchA_v, nchB_v, ones_v, zbuf_v, acc_sh):
        c = lax.axis_index("c")
        s = lax.axis_index("s")

        _fill_vmem_rows(ones_v, CH, DH, 1.0)
        _fill_vmem_rows(zbuf_v, ZR, DH, 0.0)
        _per_tile_chunks(
            s, lambda b: pltpu.sync_copy(zbuf_v, acc_sh.at[pl.ds(b, ZR)]))
        pltpu.sync_copy(nchA_hbm, nchA_v)
        pltpu.sync_copy(nchB_hbm, nchB_v)

        @pl.when(c == 0)
        def _():
            pltpu.sync_copy(dstA_hbm.at[s], idx_v)

        @pl.when(c == 1)
        def _():
            pltpu.sync_copy(dstB_hbm.at[s], idx_v)

        n = jnp.where(c == 0, _my_count(nchA_v, s), _my_count(nchB_v, s))
        plsc.subcore_barrier()

        def body(j, carry):
            pltpu.sync_copy(ones_v, acc_sh.at[idx_v.at[j]], add=True)
            return carry

        lax.fori_loop(0, n, body, 0)
        plsc.subcore_barrier()

        def drain(b, base):
            pltpu.sync_copy(acc_sh.at[pl.ds(b, ZR)],
                            out_hbm.at[pl.ds(base + b, ZR)])

        @pl.when(c == 0)
        def _():
            _per_tile_chunks(s, lambda b: drain(b, 0))

        @pl.when(c == 1)
        def _():
            _per_tile_chunks(s, lambda b: drain(b, NH))

    return k(dstA, dstB, nchA, nchB)


def _spmm_sc(x0, x1, srcA, dstA, srcB, dstB, nchA, nchB):
    @functools.partial(
        pl.kernel,
        out_type=[jax.ShapeDtypeStruct((N, DH), jnp.float32),
                  jax.ShapeDtypeStruct((N, DH), jnp.float32)],
        mesh=_mesh(),
        compiler_params=pltpu.CompilerParams(needs_layout_passes=False),
        scratch_types=[
            pltpu.VMEM((LROWS, CH), jnp.int32),
            pltpu.VMEM((LROWS, CH), jnp.int32),
            pltpu.VMEM((16 * TILES,), jnp.int32),
            pltpu.VMEM((16 * TILES,), jnp.int32),
            pltpu.VMEM((CH, DH), jnp.float32),
            pltpu.VMEM((CH, DH), jnp.float32),
            pltpu.VMEM((ZR, DH), jnp.float32),
            pltpu.VMEM_SHARED((NACC, DH), jnp.float32),
            pltpu.SemaphoreType.DMA,
            pltpu.SemaphoreType.DMA,
        ],
    )
    def k(x0_hbm, x1_hbm, srcA_hbm, dstA_hbm, srcB_hbm, dstB_hbm,
          nchA_hbm, nchB_hbm, y0_hbm, y1_hbm,
          src_v, dst_v, nchA_v, nchB_v, rb0, rb1, zbuf_v, acc_sh,
          gsem0, gsem1):
        c = lax.axis_index("c")
        s = lax.axis_index("s")

        _fill_vmem_rows(zbuf_v, ZR, DH, 0.0)
        pltpu.sync_copy(nchA_hbm, nchA_v)
        pltpu.sync_copy(nchB_hbm, nchB_v)

        def one_pass(x_hbm, src_hbm, dst_hbm, nch_v, y_hbm, out_base):
            n = _my_count(nch_v, s)
            pltpu.sync_copy(src_hbm.at[s], src_v)
            pltpu.sync_copy(dst_hbm.at[s], dst_v)
            _per_tile_chunks(
                s, lambda b: pltpu.sync_copy(zbuf_v, acc_sh.at[pl.ds(b, ZR)]))
            plsc.subcore_barrier()

            @pl.when(n >= 1)
            def _():
                pltpu.async_copy(x_hbm.at[src_v.at[0]], rb0, gsem0)

            def body(t, carry):
                j0 = 2 * t
                pltpu.async_copy(x_hbm.at[src_v.at[j0 + 1]], rb1, gsem1)
                pltpu.make_async_copy(x_hbm.at[src_v.at[j0]], rb0, gsem0
                                      ).wait()
                pltpu.sync_copy(rb0, acc_sh.at[dst_v.at[j0]], add=True)

                @pl.when(j0 + 2 < n)
                def _():
                    pltpu.async_copy(x_hbm.at[src_v.at[j0 + 2]], rb0, gsem0)

                pltpu.make_async_copy(x_hbm.at[src_v.at[j0 + 1]], rb1, gsem1
                                      ).wait()
                pltpu.sync_copy(rb1, acc_sh.at[dst_v.at[j0 + 1]], add=True)
                return carry

            lax.fori_loop(0, n // 2, body, 0)

            @pl.when(n % 2 == 1)
            def _():
                jl = n - 1
                pltpu.make_async_copy(x_hbm.at[src_v.at[jl]], rb0, gsem0
                                      ).wait()
                pltpu.sync_copy(rb0, acc_sh.at[dst_v.at[jl]], add=True)

            plsc.subcore_barrier()
            _per_tile_chunks(
                s, lambda b: pltpu.sync_copy(
                    acc_sh.at[pl.ds(b, ZR)],
                    y_hbm.at[pl.ds(out_base + b, ZR)]))

        def both_passes(x_hbm, y_hbm):
            one_pass(x_hbm, srcA_hbm, dstA_hbm, nchA_v, y_hbm, 0)
            plsc.subcore_barrier()
            one_pass(x_hbm, srcB_hbm, dstB_hbm, nchB_v, y_hbm, NH)

        @pl.when(c == 0)
        def _():
            both_passes(x0_hbm, y0_hbm)

        @pl.when(c == 1)
        def _():
            both_passes(x1_hbm, y1_hbm)

    return k(x0, x1, srcA, dstA, srcB, dstB, nchA, nchB)


def _dinv_of(deg_ref):
    return lax.rsqrt(deg_ref[:, 0:1] + 1.0)


def _half_shapes():
    return [jax.ShapeDtypeStruct((N, DH), jnp.float32) for _ in range(2)]


def _tc_first(x, w0, deg):
    def body(x_ref, w_ref, deg_ref, o0_ref, o1_ref):
        dinv = _dinv_of(deg_ref)
        xw = jnp.dot(x_ref[...], w_ref[...], preferred_element_type=jnp.float32)
        xt = dinv * xw
        o0_ref[...] = xt[:, :DH]
        o1_ref[...] = xt[:, DH:]

    half = pl.BlockSpec((RB, DH), lambda i: (i, 0))
    return pl.pallas_call(
        body,
        grid=(GRID,),
        in_specs=[
            pl.BlockSpec((RB, D), lambda i: (i, 0)),
            pl.BlockSpec((D, D), lambda i: (0, 0)),
            half,
        ],
        out_specs=[half, half],
        out_shape=_half_shapes(),
    )(x, w0, deg)


def _tc_mid(y0, y1, p0, p1, deg, b, w):
    def body(y0_ref, y1_ref, p0_ref, p1_ref, deg_ref, b_ref, w_ref,
             o0_ref, o1_ref):
        dinv = _dinv_of(deg_ref)
        h0 = jax.nn.relu(dinv * (y0_ref[...] + p0_ref[...]) + b_ref[0:1, :DH])
        h1 = jax.nn.relu(dinv * (y1_ref[...] + p1_ref[...]) + b_ref[0:1, DH:])
        h = jnp.concatenate([h0, h1], axis=1)
        xw = jnp.dot(h, w_ref[...], preferred_element_type=jnp.float32)
        xt = dinv * xw
        o0_ref[...] = xt[:, :DH]
        o1_ref[...] = xt[:, DH:]

    half = pl.BlockSpec((RB, DH), lambda i: (i, 0))
    return pl.pallas_call(
        body,
        grid=(GRID,),
        in_specs=[
            half, half, half, half, half,
            pl.BlockSpec((1, D), lambda i: (0, 0)),
            pl.BlockSpec((D, D), lambda i: (0, 0)),
        ],
        out_specs=[half, half],
        out_shape=_half_shapes(),
    )(y0, y1, p0, p1, deg, b, w)


def _tc_final(y0, y1, p0, p1, deg, b2, wnT, bn, batch3, wf1, bf1, wf2, bf2):
    def body(y0_ref, y1_ref, p0_ref, p1_ref, deg_ref, b_ref, wn_ref, bn_ref,
             batch_ref, wf1_ref, bf1_ref, wf2_ref, bf2_ref,
             node_ref, fea_ref, pooled_acc, cnt_acc):
        i = pl.program_id(0)

        @pl.when(i == 0)
        def _():
            pooled_acc[...] = jnp.zeros_like(pooled_acc)
            cnt_acc[...] = jnp.zeros_like(cnt_acc)

        dinv = _dinv_of(deg_ref)
        h0 = jax.nn.relu(dinv * (y0_ref[...] + p0_ref[...]) + b_ref[0:1, :DH])
        h1 = jax.nn.relu(dinv * (y1_ref[...] + p1_ref[...]) + b_ref[0:1, DH:])
        h = jnp.concatenate([h0, h1], axis=1)

        npv = jnp.sum(h * wn_ref[...], axis=1) + bn_ref[0, 0]
        node_ref[...] = jax.nn.sigmoid(npv).reshape(1, 1, RB)

        brow = batch_ref[0, 0, :]
        seg = lax.broadcasted_iota(jnp.int32, (G, RB), 0)
        onehot = jnp.where(seg == brow[None, :], 1.0, 0.0).astype(jnp.float32)
        pooled_acc[...] += jnp.dot(onehot, h,
                                   preferred_element_type=jnp.float32)
        cnt = jnp.sum(onehot, axis=1)
        cnt_acc[...] += jnp.broadcast_to(cnt[:, None], (G, 128))

        @pl.when(i == GRID - 1)
        def _():
            cntcol = cnt_acc[:, 0:1]
            pooled = pooled_acc[...] / jnp.maximum(cntcol, 1.0)
            f1 = jax.nn.relu(
                jnp.dot(pooled, wf1_ref[...],
                        preferred_element_type=jnp.float32) + bf1_ref[0:1, :])
            f2 = jnp.dot(f1, wf2_ref[...],
                         preferred_element_type=jnp.float32) + bf2_ref[0:1, :]
            fea_ref[...] = jax.nn.sigmoid(f2)

    half = pl.BlockSpec((RB, DH), lambda i: (i, 0))
    return pl.pallas_call(
        body,
        grid=(GRID,),
        in_specs=[
            half, half, half, half, half,
            pl.BlockSpec((1, D), lambda i: (0, 0)),
            pl.BlockSpec((1, D), lambda i: (0, 0)),
            pl.BlockSpec((1, 1), lambda i: (0, 0)),
            pl.BlockSpec((1, 1, RB), lambda i: (i, 0, 0)),
            pl.BlockSpec((D, H), lambda i: (0, 0)),
            pl.BlockSpec((1, H), lambda i: (0, 0)),
            pl.BlockSpec((H, D), lambda i: (0, 0)),
            pl.BlockSpec((1, D), lambda i: (0, 0)),
        ],
        out_specs=[
            pl.BlockSpec((1, 1, RB), lambda i: (i, 0, 0)),
            pl.BlockSpec((G, D), lambda i: (0, 0)),
        ],
        out_shape=[jax.ShapeDtypeStruct((GRID, 1, RB), jnp.float32),
                   jax.ShapeDtypeStruct((G, D), jnp.float32)],
        scratch_shapes=[
            pltpu.VMEM((G, D), jnp.float32),
            pltpu.VMEM((G, 128), jnp.float32),
        ],
    )(y0, y1, p0, p1, deg, b2, wnT, bn, batch3, wf1, bf1, wf2, bf2)


def kernel(x, edge_index, batch, W0, b0, W1, b1, W2, b2, Wn, bn,
           Wf1, bf1, Wf2, bf2):
    src3d = edge_index[0].reshape(TILES, ROWS_PER_TILE, CH)
    dst3d = edge_index[1].reshape(TILES, ROWS_PER_TILE, CH)

    srcA, dstA, srcB, dstB, nchA, nchB = _part_sc(src3d, dst3d)
    lists = (srcA, dstA, srcB, dstB, nchA, nchB)

    deg = _deg_sc(dstA, dstB, nchA, nchB)

    p0, p1 = _tc_first(x, W0, deg)
    y0, y1 = _spmm_sc(p0, p1, *lists)
    p0, p1 = _tc_mid(y0, y1, p0, p1, deg, b0.reshape(1, D), W1)
    y0, y1 = _spmm_sc(p0, p1, *lists)
    p0, p1 = _tc_mid(y0, y1, p0, p1, deg, b1.reshape(1, D), W2)
    y0, y1 = _spmm_sc(p0, p1, *lists)

    node3, fea = _tc_final(
        y0, y1, p0, p1, deg, b2.reshape(1, D), Wn.reshape(1, D),
        bn.reshape(1, 1), batch.reshape(GRID, 1, RB),
        Wf1, bf1.reshape(1, H), Wf2, bf2.reshape(1, D))

    return node3.reshape(N), fea


# speedup vs baseline: 11.5766x; 1.1028x over previous
import functools

import jax
import jax.numpy as jnp
from jax import lax
from jax.experimental import pallas as pl
from jax.experimental.pallas import tpu as pltpu
from jax.experimental.pallas import tpu_sc as plsc

N = 10000
NH = N // 2
NACC = NH + 8
D = 256
DH = 128
E = 160000
G = 64
H = 512

CH = 80
TILES = 16
EPT = E // TILES
ROWS_PER_TILE = EPT // CH

NCHUNK = 125
ZR = NH // NCHUNK

RB = 400
GRID = N // RB

EROWS = 1250
ECOLS = 128


def _mesh():
    return plsc.VectorSubcoreMesh(core_axis_name="c", subcore_axis_name="s",
                                  num_cores=2, num_subcores=16)


def _fill_vmem_rows(ref, nrows, ncols, value):
    v = jnp.full((16,), value, jnp.float32)

    def body(i, carry):
        for k in range(ncols // 16):
            ref[i, pl.ds(k * 16, 16)] = v
        return carry

    lax.fori_loop(0, nrows, body, 0)


def _per_tile_chunks(s, fn):
    for m in range((NCHUNK + TILES - 1) // TILES):
        k = s + TILES * m

        @pl.when(k < NCHUNK)
        def _(k=k):
            fn(k * ZR)


LROWS = 128


def _part_sc(src3d, dst3d):
    list_ty = jax.ShapeDtypeStruct((TILES, LROWS, CH), jnp.int32)
    cnt_ty = jax.ShapeDtypeStruct((TILES * 16,), jnp.int32)

    @functools.partial(
        pl.kernel,
        out_type=[list_ty, list_ty, list_ty, list_ty, cnt_ty, cnt_ty],
        mesh=_mesh(),
        compiler_params=pltpu.CompilerParams(needs_layout_passes=False),
        scratch_types=[
            pltpu.VMEM((ROWS_PER_TILE, CH), jnp.int32),
            pltpu.VMEM((ROWS_PER_TILE, CH), jnp.int32),
            pltpu.VMEM((LROWS, CH), jnp.int32),
            pltpu.VMEM((LROWS, CH), jnp.int32),
            pltpu.VMEM((LROWS, CH), jnp.int32),
            pltpu.VMEM((LROWS, CH), jnp.int32),
            pltpu.VMEM((16,), jnp.int32),
        ],
    )
    def k(src_hbm, dst_hbm, srcA_hbm, dstA_hbm, srcB_hbm, dstB_hbm,
          nchA_hbm, nchB_hbm, src_v, dst_v, bAs, bAd, bBs, bBd, cnt_v):
        c = lax.axis_index("c")
        s = lax.axis_index("s")

        @pl.when(c == 0)
        def _():
            pltpu.sync_copy(src_hbm.at[s], src_v)
            pltpu.sync_copy(dst_hbm.at[s], dst_v)
            iota = lax.iota(jnp.int32, 16)

            def bc(x):
                return lax.broadcast_in_dim(x, (16,), ())

            def row(j, off):
                offA, offB = off
                for kk in range(CH // 16):
                    sv = src_v[j, pl.ds(kk * 16, 16)]
                    dv = dst_v[j, pl.ds(kk * 16, 16)]
                    mA = dv < NH
                    mB = dv >= NH
                    cumA = plsc.cumsum(jnp.where(mA, 1, 0))
                    posA = bc(offA) + cumA - 1
                    posB = bc(offB) + iota - cumA
                    plsc.store_scatter(bAs, [posA // CH, posA % CH], sv,
                                       mask=mA)
                    plsc.store_scatter(bAd, [posA // CH, posA % CH], dv,
                                       mask=mA)
                    plsc.store_scatter(bBs, [posB // CH, posB % CH], sv,
                                       mask=mB)
                    plsc.store_scatter(bBd, [posB // CH, posB % CH],
                                       dv - NH, mask=mB)
                    nA = jnp.sum(jnp.where(mA, 1, 0))
                    offA = offA + nA
                    offB = offB + 16 - nA
                return (offA, offB)

            offA, offB = lax.fori_loop(0, ROWS_PER_TILE, row,
                                       (jnp.int32(0), jnp.int32(0)))

            zsrc = jnp.zeros((16,), jnp.int32)
            zdst = jnp.full((16,), NH, jnp.int32)
            mall = iota < 16
            for t in range(CH // 16):
                pA = bc(offA + t * 16) + iota
                plsc.store_scatter(bAs, [pA // CH, pA % CH], zsrc, mask=mall)
                plsc.store_scatter(bAd, [pA // CH, pA % CH], zdst, mask=mall)
                pB = bc(offB + t * 16) + iota
                plsc.store_scatter(bBs, [pB // CH, pB % CH], zsrc, mask=mall)
                plsc.store_scatter(bBd, [pB // CH, pB % CH], zdst, mask=mall)

            pltpu.sync_copy(bAs, srcA_hbm.at[s])
            pltpu.sync_copy(bAd, dstA_hbm.at[s])
            pltpu.sync_copy(bBs, srcB_hbm.at[s])
            pltpu.sync_copy(bBd, dstB_hbm.at[s])

            nchunksA = (offA + CH - 1) // CH
            cnt_v[...] = jnp.where(iota == 0, bc(nchunksA), 0)
            pltpu.sync_copy(cnt_v, nchA_hbm.at[pl.ds(s * 16, 16)])
            nchunksB = (offB + CH - 1) // CH
            cnt_v[...] = jnp.where(iota == 0, bc(nchunksB), 0)
            pltpu.sync_copy(cnt_v, nchB_hbm.at[pl.ds(s * 16, 16)])

    return k(src3d, dst3d)


def _my_count(nch_v, s):
    vec = nch_v[pl.ds(s * 16, 16)]
    iota = lax.iota(jnp.int32, 16)
    return jnp.sum(jnp.where(iota == 0, vec, 0))


def _deg_sc(dstA, dstB, nchA, nchB):
    @functools.partial(
        pl.kernel,
        out_type=jax.ShapeDtypeStruct((N, DH), jnp.float32),
        mesh=_mesh(),
        compiler_params=pltpu.CompilerParams(needs_layout_passes=False),
        scratch_types=[
            pltpu.VMEM((LROWS, CH), jnp.int32),
            pltpu.VMEM((16 * TILES,), jnp.int32),
            pltpu.VMEM((16 * TILES,), jnp.int32),
            pltpu.VMEM((CH, DH), jnp.float32),
            pltpu.VMEM((ZR, DH), jnp.float32),
            pltpu.VMEM_SHARED((NACC, DH), jnp.float32),
            pltpu.SemaphoreType.DMA,
        ],
    )
    def k(dstA_hbm, dstB_hbm, nchA_hbm, nchB_hbm, out_hbm,
          idx_v, nchA_v, nchB_v, ones_v, zbuf_v, acc_sh, ssem):
        c = lax.axis_index("c")
        s = lax.axis_index("s")

        _fill_vmem_rows(ones_v, CH, DH, 1.0)
        _fill_vmem_rows(zbuf_v, ZR, DH, 0.0)
        _per_tile_chunks(
            s, lambda b: pltpu.sync_copy(zbuf_v, acc_sh.at[pl.ds(b, ZR)]))
        pltpu.sync_copy(nchA_hbm, nchA_v)
        pltpu.sync_copy(nchB_hbm, nchB_v)

        @pl.when(c == 0)
        def _():
            pltpu.sync_copy(dstA_hbm.at[s], idx_v)

        @pl.when(c == 1)
        def _():
            pltpu.sync_copy(dstB_hbm.at[s], idx_v)

        n = jnp.where(c == 0, _my_count(nchA_v, s), _my_count(nchB_v, s))
        plsc.subcore_barrier()

        def body(j, carry):
            pltpu.async_copy(ones_v, acc_sh.at[idx_v.at[j]], ssem, add=True)
            return carry

        lax.fori_loop(0, n, body, 0)

        def sdrain(j, carry):
            pltpu.make_async_copy(ones_v, acc_sh.at[idx_v.at[0]], ssem).wait()
            return carry

        lax.fori_loop(0, n, sdrain, 0)
        plsc.subcore_barrier()

        def drain(b, base):
            pltpu.sync_copy(acc_sh.at[pl.ds(b, ZR)],
                            out_hbm.at[pl.ds(base + b, ZR)])

        @pl.when(c == 0)
        def _():
            _per_tile_chunks(s, lambda b: drain(b, 0))

        @pl.when(c == 1)
        def _():
            _per_tile_chunks(s, lambda b: drain(b, NH))

    return k(dstA, dstB, nchA, nchB)


def _spmm_sc(x0, x1, srcA, dstA, srcB, dstB, nchA, nchB):
    @functools.partial(
        pl.kernel,
        out_type=[jax.ShapeDtypeStruct((N, DH), jnp.float32),
                  jax.ShapeDtypeStruct((N, DH), jnp.float32)],
        mesh=_mesh(),
        compiler_params=pltpu.CompilerParams(needs_layout_passes=False),
        scratch_types=[
            pltpu.VMEM((LROWS, CH), jnp.int32),
            pltpu.VMEM((LROWS, CH), jnp.int32),
            pltpu.VMEM((16 * TILES,), jnp.int32),
            pltpu.VMEM((16 * TILES,), jnp.int32),
            pltpu.VMEM((CH, DH), jnp.float32),
            pltpu.VMEM((CH, DH), jnp.float32),
            pltpu.VMEM((CH, DH), jnp.float32),
            pltpu.VMEM((CH, DH), jnp.float32),
            pltpu.VMEM((ZR, DH), jnp.float32),
            pltpu.VMEM_SHARED((NACC, DH), jnp.float32),
            pltpu.SemaphoreType.DMA,
            pltpu.SemaphoreType.DMA,
        ],
    )
    def k(x0_hbm, x1_hbm, srcA_hbm, dstA_hbm, srcB_hbm, dstB_hbm,
          nchA_hbm, nchB_hbm, y0_hbm, y1_hbm,
          src_v, dst_v, nchA_v, nchB_v, rb0, rb1, rb2, rb3, zbuf_v, acc_sh,
          gsem, ssem):
        c = lax.axis_index("c")
        s = lax.axis_index("s")
        rbufs = (rb0, rb1, rb2, rb3)

        _fill_vmem_rows(zbuf_v, ZR, DH, 0.0)
        pltpu.sync_copy(nchA_hbm, nchA_v)
        pltpu.sync_copy(nchB_hbm, nchB_v)

        def one_pass(x_hbm, src_hbm, dst_hbm, nch_v, y_hbm, out_base):
            n = _my_count(nch_v, s)
            pltpu.sync_copy(src_hbm.at[s], src_v)
            pltpu.sync_copy(dst_hbm.at[s], dst_v)
            _per_tile_chunks(
                s, lambda b: pltpu.sync_copy(zbuf_v, acc_sh.at[pl.ds(b, ZR)]))
            plsc.subcore_barrier()

            def waitG(buf):
                pltpu.make_async_copy(x_hbm.at[src_v.at[0]], buf, gsem).wait()

            def waitS(buf):
                pltpu.make_async_copy(buf, acc_sh.at[dst_v.at[0]], ssem).wait()

            for i in range(4):
                @pl.when(i < n)
                def _(i=i):
                    pltpu.async_copy(x_hbm.at[src_v.at[i]], rbufs[i], gsem)

            def body(q, carry):
                j = 4 * q
                for i in range(4):
                    ch = j + i
                    waitG(rbufs[i])
                    pltpu.async_copy(rbufs[i], acc_sh.at[dst_v.at[ch]], ssem,
                                     add=True)

                    @pl.when(ch + 4 < n)
                    def _(i=i, ch=ch):
                        waitS(rbufs[i])
                        pltpu.async_copy(x_hbm.at[src_v.at[ch + 4]], rbufs[i],
                                         gsem)
                return carry

            lax.fori_loop(0, n // 4, body, 0)

            tail = (n // 4) * 4
            for i in range(3):
                @pl.when(tail + i < n)
                def _(i=i):
                    waitG(rbufs[i])
                    pltpu.async_copy(rbufs[i],
                                     acc_sh.at[dst_v.at[tail + i]], ssem,
                                     add=True)

            def sdrain(q, carry):
                waitS(rb0)
                return carry

            lax.fori_loop(0, jnp.minimum(n, 4), sdrain, 0)
            plsc.subcore_barrier()
            _per_tile_chunks(
                s, lambda b: pltpu.sync_copy(
                    acc_sh.at[pl.ds(b, ZR)],
                    y_hbm.at[pl.ds(out_base + b, ZR)]))

        def both_passes(x_hbm, y_hbm):
            one_pass(x_hbm, srcA_hbm, dstA_hbm, nchA_v, y_hbm, 0)
            plsc.subcore_barrier()
            one_pass(x_hbm, srcB_hbm, dstB_hbm, nchB_v, y_hbm, NH)

        @pl.when(c == 0)
        def _():
            both_passes(x0_hbm, y0_hbm)

        @pl.when(c == 1)
        def _():
            both_passes(x1_hbm, y1_hbm)

    return k(x0, x1, srcA, dstA, srcB, dstB, nchA, nchB)


def _dinv_of(deg_ref):
    return lax.rsqrt(deg_ref[:, 0:1] + 1.0)


def _half_shapes():
    return [jax.ShapeDtypeStruct((N, DH), jnp.float32) for _ in range(2)]


def _tc_first(x, w0, deg):
    def body(x_ref, w_ref, deg_ref, o0_ref, o1_ref):
        dinv = _dinv_of(deg_ref)
        xw = jnp.dot(x_ref[...], w_ref[...], preferred_element_type=jnp.float32)
        xt = dinv * xw
        o0_ref[...] = xt[:, :DH]
        o1_ref[...] = xt[:, DH:]

    half = pl.BlockSpec((RB, DH), lambda i: (i, 0))
    return pl.pallas_call(
        body,
        grid=(GRID,),
        in_specs=[
            pl.BlockSpec((RB, D), lambda i: (i, 0)),
            pl.BlockSpec((D, D), lambda i: (0, 0)),
            half,
        ],
        out_specs=[half, half],
        out_shape=_half_shapes(),
    )(x, w0, deg)


def _tc_mid(y0, y1, p0, p1, deg, b, w):
    def body(y0_ref, y1_ref, p0_ref, p1_ref, deg_ref, b_ref, w_ref,
             o0_ref, o1_ref):
        dinv = _dinv_of(deg_ref)
        h0 = jax.nn.relu(dinv * (y0_ref[...] + p0_ref[...]) + b_ref[0:1, :DH])
        h1 = jax.nn.relu(dinv * (y1_ref[...] + p1_ref[...]) + b_ref[0:1, DH:])
        h = jnp.concatenate([h0, h1], axis=1)
        xw = jnp.dot(h, w_ref[...], preferred_element_type=jnp.float32)
        xt = dinv * xw
        o0_ref[...] = xt[:, :DH]
        o1_ref[...] = xt[:, DH:]

    half = pl.BlockSpec((RB, DH), lambda i: (i, 0))
    return pl.pallas_call(
        body,
        grid=(GRID,),
        in_specs=[
            half, half, half, half, half,
            pl.BlockSpec((1, D), lambda i: (0, 0)),
            pl.BlockSpec((D, D), lambda i: (0, 0)),
        ],
        out_specs=[half, half],
        out_shape=_half_shapes(),
    )(y0, y1, p0, p1, deg, b, w)


def _tc_final(y0, y1, p0, p1, deg, b2, wnT, bn, batch3, wf1, bf1, wf2, bf2):
    def body(y0_ref, y1_ref, p0_ref, p1_ref, deg_ref, b_ref, wn_ref, bn_ref,
             batch_ref, wf1_ref, bf1_ref, wf2_ref, bf2_ref,
             node_ref, fea_ref, pooled_acc, cnt_acc):
        i = pl.program_id(0)

        @pl.when(i == 0)
        def _():
            pooled_acc[...] = jnp.zeros_like(pooled_acc)
            cnt_acc[...] = jnp.zeros_like(cnt_acc)

        dinv = _dinv_of(deg_ref)
        h0 = jax.nn.relu(dinv * (y0_ref[...] + p0_ref[...]) + b_ref[0:1, :DH])
        h1 = jax.nn.relu(dinv * (y1_ref[...] + p1_ref[...]) + b_ref[0:1, DH:])
        h = jnp.concatenate([h0, h1], axis=1)

        npv = jnp.sum(h * wn_ref[...], axis=1) + bn_ref[0, 0]
        node_ref[...] = jax.nn.sigmoid(npv).reshape(1, 1, RB)

        brow = batch_ref[0, 0, :]
        seg = lax.broadcasted_iota(jnp.int32, (G, RB), 0)
        onehot = jnp.where(seg == brow[None, :], 1.0, 0.0).astype(jnp.float32)
        pooled_acc[...] += jnp.dot(onehot, h,
                                   preferred_element_type=jnp.float32)
        cnt = jnp.sum(onehot, axis=1)
        cnt_acc[...] += jnp.broadcast_to(cnt[:, None], (G, 128))

        @pl.when(i == GRID - 1)
        def _():
            cntcol = cnt_acc[:, 0:1]
            pooled = pooled_acc[...] / jnp.maximum(cntcol, 1.0)
            f1 = jax.nn.relu(
                jnp.dot(pooled, wf1_ref[...],
                        preferred_element_type=jnp.float32) + bf1_ref[0:1, :])
            f2 = jnp.dot(f1, wf2_ref[...],
                         preferred_element_type=jnp.float32) + bf2_ref[0:1, :]
            fea_ref[...] = jax.nn.sigmoid(f2)

    half = pl.BlockSpec((RB, DH), lambda i: (i, 0))
    return pl.pallas_call(
        body,
        grid=(GRID,),
        in_specs=[
            half, half, half, half, half,
            pl.BlockSpec((1, D), lambda i: (0, 0)),
            pl.BlockSpec((1, D), lambda i: (0, 0)),
            pl.BlockSpec((1, 1), lambda i: (0, 0)),
            pl.BlockSpec((1, 1, RB), lambda i: (i, 0, 0)),
            pl.BlockSpec((D, H), lambda i: (0, 0)),
            pl.BlockSpec((1, H), lambda i: (0, 0)),
            pl.BlockSpec((H, D), lambda i: (0, 0)),
            pl.BlockSpec((1, D), lambda i: (0, 0)),
        ],
        out_specs=[
            pl.BlockSpec((1, 1, RB), lambda i: (i, 0, 0)),
            pl.BlockSpec((G, D), lambda i: (0, 0)),
        ],
        out_shape=[jax.ShapeDtypeStruct((GRID, 1, RB), jnp.float32),
                   jax.ShapeDtypeStruct((G, D), jnp.float32)],
        scratch_shapes=[
            pltpu.VMEM((G, D), jnp.float32),
            pltpu.VMEM((G, 128), jnp.float32),
        ],
    )(y0, y1, p0, p1, deg, b2, wnT, bn, batch3, wf1, bf1, wf2, bf2)


def kernel(x, edge_index, batch, W0, b0, W1, b1, W2, b2, Wn, bn,
           Wf1, bf1, Wf2, bf2):
    src3d = edge_index[0].reshape(TILES, ROWS_PER_TILE, CH)
    dst3d = edge_index[1].reshape(TILES, ROWS_PER_TILE, CH)

    srcA, dstA, srcB, dstB, nchA, nchB = _part_sc(src3d, dst3d)
    lists = (srcA, dstA, srcB, dstB, nchA, nchB)

    deg = _deg_sc(dstA, dstB, nchA, nchB)

    p0, p1 = _tc_first(x, W0, deg)
    y0, y1 = _spmm_sc(p0, p1, *lists)
    p0, p1 = _tc_mid(y0, y1, p0, p1, deg, b0.reshape(1, D), W1)
    y0, y1 = _spmm_sc(p0, p1, *lists)
    p0, p1 = _tc_mid(y0, y1, p0, p1, deg, b1.reshape(1, D), W2)
    y0, y1 = _spmm_sc(p0, p1, *lists)

    node3, fea = _tc_final(
        y0, y1, p0, p1, deg, b2.reshape(1, D), Wn.reshape(1, D),
        bn.reshape(1, 1), batch.reshape(GRID, 1, RB),
        Wf1, bf1.reshape(1, H), Wf2, bf2.reshape(1, D))

    return node3.reshape(N), fea


# speedup vs baseline: 11.6054x; 1.0025x over previous
import functools

import jax
import jax.numpy as jnp
from jax import lax
from jax.experimental import pallas as pl
from jax.experimental.pallas import tpu as pltpu
from jax.experimental.pallas import tpu_sc as plsc

N = 10000
NH = N // 2
NACC = NH + 8
D = 256
DH = 128
E = 160000
G = 64
H = 512

CH = 80
NBUF = 4
TILES = 16
EPT = E // TILES
ROWS_PER_TILE = EPT // CH

NCHUNK = 125
ZR = NH // NCHUNK

RB = 400
GRID = N // RB

EROWS = 1250
ECOLS = 128


def _mesh():
    return plsc.VectorSubcoreMesh(core_axis_name="c", subcore_axis_name="s",
                                  num_cores=2, num_subcores=16)


def _fill_vmem_rows(ref, nrows, ncols, value):
    v = jnp.full((16,), value, jnp.float32)

    def body(i, carry):
        for k in range(ncols // 16):
            ref[i, pl.ds(k * 16, 16)] = v
        return carry

    lax.fori_loop(0, nrows, body, 0)


def _per_tile_chunks(s, fn):
    for m in range((NCHUNK + TILES - 1) // TILES):
        k = s + TILES * m

        @pl.when(k < NCHUNK)
        def _(k=k):
            fn(k * ZR)


LROWS = 128


def _part_sc(src3d, dst3d):
    list_ty = jax.ShapeDtypeStruct((TILES, LROWS, CH), jnp.int32)
    cnt_ty = jax.ShapeDtypeStruct((TILES * 16,), jnp.int32)

    @functools.partial(
        pl.kernel,
        out_type=[list_ty, list_ty, list_ty, list_ty, cnt_ty, cnt_ty],
        mesh=_mesh(),
        compiler_params=pltpu.CompilerParams(needs_layout_passes=False),
        scratch_types=[
            pltpu.VMEM((ROWS_PER_TILE, CH), jnp.int32),
            pltpu.VMEM((ROWS_PER_TILE, CH), jnp.int32),
            pltpu.VMEM((LROWS, CH), jnp.int32),
            pltpu.VMEM((LROWS, CH), jnp.int32),
            pltpu.VMEM((LROWS, CH), jnp.int32),
            pltpu.VMEM((LROWS, CH), jnp.int32),
            pltpu.VMEM((16,), jnp.int32),
        ],
    )
    def k(src_hbm, dst_hbm, srcA_hbm, dstA_hbm, srcB_hbm, dstB_hbm,
          nchA_hbm, nchB_hbm, src_v, dst_v, bAs, bAd, bBs, bBd, cnt_v):
        c = lax.axis_index("c")
        s = lax.axis_index("s")

        @pl.when(c == 0)
        def _():
            pltpu.sync_copy(src_hbm.at[s], src_v)
            pltpu.sync_copy(dst_hbm.at[s], dst_v)
            iota = lax.iota(jnp.int32, 16)

            def row(j, off):
                offA, offB = off
                for kk in range(CH // 16):
                    sv = src_v[j, pl.ds(kk * 16, 16)]
                    dv = dst_v[j, pl.ds(kk * 16, 16)]
                    mA = dv < NH
                    mB = dv >= NH
                    cumA = plsc.cumsum(jnp.where(mA, 1, 0))
                    posA = offA + cumA - 1
                    posB = offB + iota - cumA
                    plsc.store_scatter(bAs, [posA // CH, posA % CH], sv,
                                       mask=mA)
                    plsc.store_scatter(bAd, [posA // CH, posA % CH], dv,
                                       mask=mA)
                    plsc.store_scatter(bBs, [posB // CH, posB % CH], sv,
                                       mask=mB)
                    plsc.store_scatter(bBd, [posB // CH, posB % CH],
                                       dv - NH, mask=mB)
                    nA = plsc.all_reduce_population_count(mA)
                    offA = offA + nA
                    offB = offB + 16 - nA
                return (offA, offB)

            zeros16 = jnp.zeros((16,), jnp.int32)
            offA, offB = lax.fori_loop(0, ROWS_PER_TILE, row,
                                       (zeros16, zeros16))

            zsrc = jnp.zeros((16,), jnp.int32)
            zdst = jnp.full((16,), NH, jnp.int32)
            mall = iota < 16
            for t in range(CH // 16):
                pA = offA + t * 16 + iota
                plsc.store_scatter(bAs, [pA // CH, pA % CH], zsrc, mask=mall)
                plsc.store_scatter(bAd, [pA // CH, pA % CH], zdst, mask=mall)
                pB = offB + t * 16 + iota
                plsc.store_scatter(bBs, [pB // CH, pB % CH], zsrc, mask=mall)
                plsc.store_scatter(bBd, [pB // CH, pB % CH], zdst, mask=mall)

            pltpu.sync_copy(bAs, srcA_hbm.at[s])
            pltpu.sync_copy(bAd, dstA_hbm.at[s])
            pltpu.sync_copy(bBs, srcB_hbm.at[s])
            pltpu.sync_copy(bBd, dstB_hbm.at[s])

            nchunksA = (offA + CH - 1) // CH
            cnt_v[...] = jnp.where(iota == 0, nchunksA, 0)
            pltpu.sync_copy(cnt_v, nchA_hbm.at[pl.ds(s * 16, 16)])
            nchunksB = (offB + CH - 1) // CH
            cnt_v[...] = jnp.where(iota == 0, nchunksB, 0)
            pltpu.sync_copy(cnt_v, nchB_hbm.at[pl.ds(s * 16, 16)])

    return k(src3d, dst3d)


def _my_count(nch_v, s):
    vec = nch_v[pl.ds(s * 16, 16)]
    iota = lax.iota(jnp.int32, 16)
    return jnp.sum(jnp.where(iota == 0, vec, 0))


def _deg_sc(dstA, dstB, nchA, nchB):
    @functools.partial(
        pl.kernel,
        out_type=jax.ShapeDtypeStruct((N, DH), jnp.float32),
        mesh=_mesh(),
        compiler_params=pltpu.CompilerParams(needs_layout_passes=False),
        scratch_types=[
            pltpu.VMEM((LROWS, CH), jnp.int32),
            pltpu.VMEM((16 * TILES,), jnp.int32),
            pltpu.VMEM((16 * TILES,), jnp.int32),
            pltpu.VMEM((CH, DH), jnp.float32),
            pltpu.VMEM((ZR, DH), jnp.float32),
            pltpu.VMEM_SHARED((NACC, DH), jnp.float32),
            pltpu.SemaphoreType.DMA,
        ],
    )
    def k(dstA_hbm, dstB_hbm, nchA_hbm, nchB_hbm, out_hbm,
          idx_v, nchA_v, nchB_v, ones_v, zbuf_v, acc_sh, ssem):
        c = lax.axis_index("c")
        s = lax.axis_index("s")

        _fill_vmem_rows(ones_v, CH, DH, 1.0)
        _fill_vmem_rows(zbuf_v, ZR, DH, 0.0)
        _per_tile_chunks(
            s, lambda b: pltpu.sync_copy(zbuf_v, acc_sh.at[pl.ds(b, ZR)]))
        pltpu.sync_copy(nchA_hbm, nchA_v)
        pltpu.sync_copy(nchB_hbm, nchB_v)

        @pl.when(c == 0)
        def _():
            pltpu.sync_copy(dstA_hbm.at[s], idx_v)

        @pl.when(c == 1)
        def _():
            pltpu.sync_copy(dstB_hbm.at[s], idx_v)

        n = jnp.where(c == 0, _my_count(nchA_v, s), _my_count(nchB_v, s))
        plsc.subcore_barrier()

        def body(j, carry):
            pltpu.async_copy(ones_v, acc_sh.at[idx_v.at[j]], ssem, add=True)
            return carry

        lax.fori_loop(0, n, body, 0)

        def sdrain(j, carry):
            pltpu.make_async_copy(ones_v, acc_sh.at[idx_v.at[0]], ssem).wait()
            return carry

        lax.fori_loop(0, n, sdrain, 0)
        plsc.subcore_barrier()

        def drain(b, base):
            pltpu.sync_copy(acc_sh.at[pl.ds(b, ZR)],
                            out_hbm.at[pl.ds(base + b, ZR)])

        @pl.when(c == 0)
        def _():
            _per_tile_chunks(s, lambda b: drain(b, 0))

        @pl.when(c == 1)
        def _():
            _per_tile_chunks(s, lambda b: drain(b, NH))

    return k(dstA, dstB, nchA, nchB)


def _spmm_sc(x0, x1, srcA, dstA, srcB, dstB, nchA, nchB):
    @functools.partial(
        pl.kernel,
        out_type=[jax.ShapeDtypeStruct((N, DH), jnp.float32),
                  jax.ShapeDtypeStruct((N, DH), jnp.float32)],
        mesh=_mesh(),
        compiler_params=pltpu.CompilerParams(needs_layout_passes=False),
        scratch_types=[
            pltpu.VMEM((LROWS, CH), jnp.int32),
            pltpu.VMEM((LROWS, CH), jnp.int32),
            pltpu.VMEM((16 * TILES,), jnp.int32),
            pltpu.VMEM((16 * TILES,), jnp.int32),
        ] + [pltpu.VMEM((CH, DH), jnp.float32)] * NBUF + [
            pltpu.VMEM((ZR, DH), jnp.float32),
            pltpu.VMEM_SHARED((NACC, DH), jnp.float32),
            pltpu.SemaphoreType.DMA,
            pltpu.SemaphoreType.DMA,
        ],
    )
    def k(x0_hbm, x1_hbm, srcA_hbm, dstA_hbm, srcB_hbm, dstB_hbm,
          nchA_hbm, nchB_hbm, y0_hbm, y1_hbm,
          src_v, dst_v, nchA_v, nchB_v, *rest):
        rbufs = rest[:NBUF]
        zbuf_v, acc_sh, gsem, ssem = rest[NBUF:]
        c = lax.axis_index("c")
        s = lax.axis_index("s")
        rb0 = rbufs[0]

        _fill_vmem_rows(zbuf_v, ZR, DH, 0.0)
        pltpu.sync_copy(nchA_hbm, nchA_v)
        pltpu.sync_copy(nchB_hbm, nchB_v)

        def one_pass(x_hbm, src_hbm, dst_hbm, nch_v, y_hbm, out_base):
            n = _my_count(nch_v, s)
            pltpu.sync_copy(src_hbm.at[s], src_v)
            pltpu.sync_copy(dst_hbm.at[s], dst_v)
            _per_tile_chunks(
                s, lambda b: pltpu.sync_copy(zbuf_v, acc_sh.at[pl.ds(b, ZR)]))
            plsc.subcore_barrier()

            def waitG(buf):
                pltpu.make_async_copy(x_hbm.at[src_v.at[0]], buf, gsem).wait()

            def waitS(buf):
                pltpu.make_async_copy(buf, acc_sh.at[dst_v.at[0]], ssem).wait()

            for i in range(NBUF):
                @pl.when(i < n)
                def _(i=i):
                    pltpu.async_copy(x_hbm.at[src_v.at[i]], rbufs[i], gsem)

            def body(q, carry):
                j = NBUF * q
                for i in range(NBUF):
                    ch = j + i
                    waitG(rbufs[i])
                    pltpu.async_copy(rbufs[i], acc_sh.at[dst_v.at[ch]], ssem,
                                     add=True)

                    @pl.when(ch + NBUF < n)
                    def _(i=i, ch=ch):
                        waitS(rbufs[i])
                        pltpu.async_copy(x_hbm.at[src_v.at[ch + NBUF]],
                                         rbufs[i], gsem)
                return carry

            lax.fori_loop(0, n // NBUF, body, 0)

            tail = (n // NBUF) * NBUF
            for i in range(NBUF - 1):
                @pl.when(tail + i < n)
                def _(i=i):
                    waitG(rbufs[i])
                    pltpu.async_copy(rbufs[i],
                                     acc_sh.at[dst_v.at[tail + i]], ssem,
                                     add=True)

            def sdrain(q, carry):
                waitS(rb0)
                return carry

            lax.fori_loop(0, jnp.minimum(n, NBUF), sdrain, 0)
            plsc.subcore_barrier()
            _per_tile_chunks(
                s, lambda b: pltpu.sync_copy(
                    acc_sh.at[pl.ds(b, ZR)],
                    y_hbm.at[pl.ds(out_base + b, ZR)]))

        def both_passes(x_hbm, y_hbm):
            one_pass(x_hbm, srcA_hbm, dstA_hbm, nchA_v, y_hbm, 0)
            plsc.subcore_barrier()
            one_pass(x_hbm, srcB_hbm, dstB_hbm, nchB_v, y_hbm, NH)

        @pl.when(c == 0)
        def _():
            both_passes(x0_hbm, y0_hbm)

        @pl.when(c == 1)
        def _():
            both_passes(x1_hbm, y1_hbm)

    return k(x0, x1, srcA, dstA, srcB, dstB, nchA, nchB)


def _dinv_of(deg_ref):
    return lax.rsqrt(deg_ref[:, 0:1] + 1.0)


def _half_shapes():
    return [jax.ShapeDtypeStruct((N, DH), jnp.float32) for _ in range(2)]


def _tc_first(x, w0, deg):
    def body(x_ref, w_ref, deg_ref, o0_ref, o1_ref):
        dinv = _dinv_of(deg_ref)
        xw = jnp.dot(x_ref[...], w_ref[...], preferred_element_type=jnp.float32)
        xt = dinv * xw
        o0_ref[...] = xt[:, :DH]
        o1_ref[...] = xt[:, DH:]

    half = pl.BlockSpec((RB, DH), lambda i: (i, 0))
    return pl.pallas_call(
        body,
        grid=(GRID,),
        in_specs=[
            pl.BlockSpec((RB, D), lambda i: (i, 0)),
            pl.BlockSpec((D, D), lambda i: (0, 0)),
            half,
        ],
        out_specs=[half, half],
        out_shape=_half_shapes(),
    )(x, w0, deg)


def _tc_mid(y0, y1, p0, p1, deg, b, w):
    def body(y0_ref, y1_ref, p0_ref, p1_ref, deg_ref, b_ref, w_ref,
             o0_ref, o1_ref):
        dinv = _dinv_of(deg_ref)
        h0 = jax.nn.relu(dinv * (y0_ref[...] + p0_ref[...]) + b_ref[0:1, :DH])
        h1 = jax.nn.relu(dinv * (y1_ref[...] + p1_ref[...]) + b_ref[0:1, DH:])
        h = jnp.concatenate([h0, h1], axis=1)
        xw = jnp.dot(h, w_ref[...], preferred_element_type=jnp.float32)
        xt = dinv * xw
        o0_ref[...] = xt[:, :DH]
        o1_ref[...] = xt[:, DH:]

    half = pl.BlockSpec((RB, DH), lambda i: (i, 0))
    return pl.pallas_call(
        body,
        grid=(GRID,),
        in_specs=[
            half, half, half, half, half,
            pl.BlockSpec((1, D), lambda i: (0, 0)),
            pl.BlockSpec((D, D), lambda i: (0, 0)),
        ],
        out_specs=[half, half],
        out_shape=_half_shapes(),
    )(y0, y1, p0, p1, deg, b, w)


def _tc_final(y0, y1, p0, p1, deg, b2, wnT, bn, batch3, wf1, bf1, wf2, bf2):
    def body(y0_ref, y1_ref, p0_ref, p1_ref, deg_ref, b_ref, wn_ref, bn_ref,
             batch_ref, wf1_ref, bf1_ref, wf2_ref, bf2_ref,
             node_ref, fea_ref, pooled_acc, cnt_acc):
        i = pl.program_id(0)

        @pl.when(i == 0)
        def _():
            pooled_acc[...] = jnp.zeros_like(pooled_acc)
            cnt_acc[...] = jnp.zeros_like(cnt_acc)

        dinv = _dinv_of(deg_ref)
        h0 = jax.nn.relu(dinv * (y0_ref[...] + p0_ref[...]) + b_ref[0:1, :DH])
        h1 = jax.nn.relu(dinv * (y1_ref[...] + p1_ref[...]) + b_ref[0:1, DH:])
        h = jnp.concatenate([h0, h1], axis=1)

        npv = jnp.sum(h * wn_ref[...], axis=1) + bn_ref[0, 0]
        node_ref[...] = jax.nn.sigmoid(npv).reshape(1, 1, RB)

        brow = batch_ref[0, 0, :]
        seg = lax.broadcasted_iota(jnp.int32, (G, RB), 0)
        onehot = jnp.where(seg == brow[None, :], 1.0, 0.0).astype(jnp.float32)
        pooled_acc[...] += jnp.dot(onehot, h,
                                   preferred_element_type=jnp.float32)
        cnt = jnp.sum(onehot, axis=1)
        cnt_acc[...] += jnp.broadcast_to(cnt[:, None], (G, 128))

        @pl.when(i == GRID - 1)
        def _():
            cntcol = cnt_acc[:, 0:1]
            pooled = pooled_acc[...] / jnp.maximum(cntcol, 1.0)
            f1 = jax.nn.relu(
                jnp.dot(pooled, wf1_ref[...],
                        preferred_element_type=jnp.float32) + bf1_ref[0:1, :])
            f2 = jnp.dot(f1, wf2_ref[...],
                         preferred_element_type=jnp.float32) + bf2_ref[0:1, :]
            fea_ref[...] = jax.nn.sigmoid(f2)

    half = pl.BlockSpec((RB, DH), lambda i: (i, 0))
    return pl.pallas_call(
        body,
        grid=(GRID,),
        in_specs=[
            half, half, half, half, half,
            pl.BlockSpec((1, D), lambda i: (0, 0)),
            pl.BlockSpec((1, D), lambda i: (0, 0)),
            pl.BlockSpec((1, 1), lambda i: (0, 0)),
            pl.BlockSpec((1, 1, RB), lambda i: (i, 0, 0)),
            pl.BlockSpec((D, H), lambda i: (0, 0)),
            pl.BlockSpec((1, H), lambda i: (0, 0)),
            pl.BlockSpec((H, D), lambda i: (0, 0)),
            pl.BlockSpec((1, D), lambda i: (0, 0)),
        ],
        out_specs=[
            pl.BlockSpec((1, 1, RB), lambda i: (i, 0, 0)),
            pl.BlockSpec((G, D), lambda i: (0, 0)),
        ],
        out_shape=[jax.ShapeDtypeStruct((GRID, 1, RB), jnp.float32),
                   jax.ShapeDtypeStruct((G, D), jnp.float32)],
        scratch_shapes=[
            pltpu.VMEM((G, D), jnp.float32),
            pltpu.VMEM((G, 128), jnp.float32),
        ],
    )(y0, y1, p0, p1, deg, b2, wnT, bn, batch3, wf1, bf1, wf2, bf2)


def kernel(x, edge_index, batch, W0, b0, W1, b1, W2, b2, Wn, bn,
           Wf1, bf1, Wf2, bf2):
    src3d = edge_index[0].reshape(TILES, ROWS_PER_TILE, CH)
    dst3d = edge_index[1].reshape(TILES, ROWS_PER_TILE, CH)

    srcA, dstA, srcB, dstB, nchA, nchB = _part_sc(src3d, dst3d)
    lists = (srcA, dstA, srcB, dstB, nchA, nchB)

    deg = _deg_sc(dstA, dstB, nchA, nchB)

    p0, p1 = _tc_first(x, W0, deg)
    y0, y1 = _spmm_sc(p0, p1, *lists)
    p0, p1 = _tc_mid(y0, y1, p0, p1, deg, b0.reshape(1, D), W1)
    y0, y1 = _spmm_sc(p0, p1, *lists)
    p0, p1 = _tc_mid(y0, y1, p0, p1, deg, b1.reshape(1, D), W2)
    y0, y1 = _spmm_sc(p0, p1, *lists)

    node3, fea = _tc_final(
        y0, y1, p0, p1, deg, b2.reshape(1, D), Wn.reshape(1, D),
        bn.reshape(1, 1), batch.reshape(GRID, 1, RB),
        Wf1, bf1.reshape(1, H), Wf2, bf2.reshape(1, D))

    return node3.reshape(N), fea


# speedup vs baseline: 11.6066x; 1.0001x over previous
import functools

import jax
import jax.numpy as jnp
from jax import lax
from jax.experimental import pallas as pl
from jax.experimental.pallas import tpu as pltpu
from jax.experimental.pallas import tpu_sc as plsc

N = 10000
NH = N // 2
NACC = NH + 8
D = 256
DH = 128
E = 160000
G = 64
H = 512

CH = 80
NBUF = 4
TILES = 16
EPT = E // TILES
ROWS_PER_TILE = EPT // CH

NCHUNK = 125
ZR = NH // NCHUNK

RB = 400
GRID = N // RB

EROWS = 1250
ECOLS = 128


def _mesh():
    return plsc.VectorSubcoreMesh(core_axis_name="c", subcore_axis_name="s",
                                  num_cores=2, num_subcores=16)


def _fill_vmem_rows(ref, nrows, ncols, value):
    v = jnp.full((16,), value, jnp.float32)

    def body(i, carry):
        for k in range(ncols // 16):
            ref[i, pl.ds(k * 16, 16)] = v
        return carry

    lax.fori_loop(0, nrows, body, 0)


def _per_tile_chunks(s, fn):
    for m in range((NCHUNK + TILES - 1) // TILES):
        k = s + TILES * m

        @pl.when(k < NCHUNK)
        def _(k=k):
            fn(k * ZR)


LROWS = 128


def _part_sc(src3d, dst3d):
    list_ty = jax.ShapeDtypeStruct((TILES, LROWS, CH), jnp.int32)
    cnt_ty = jax.ShapeDtypeStruct((TILES * 16,), jnp.int32)

    @functools.partial(
        pl.kernel,
        out_type=[list_ty, list_ty, list_ty, list_ty, cnt_ty, cnt_ty],
        mesh=_mesh(),
        compiler_params=pltpu.CompilerParams(needs_layout_passes=False),
        scratch_types=[
            pltpu.VMEM((ROWS_PER_TILE, CH), jnp.int32),
            pltpu.VMEM((ROWS_PER_TILE, CH), jnp.int32),
            pltpu.VMEM((LROWS, CH), jnp.int32),
            pltpu.VMEM((LROWS, CH), jnp.int32),
            pltpu.VMEM((LROWS, CH), jnp.int32),
            pltpu.VMEM((LROWS, CH), jnp.int32),
            pltpu.VMEM((16,), jnp.int32),
        ],
    )
    def k(src_hbm, dst_hbm, srcA_hbm, dstA_hbm, srcB_hbm, dstB_hbm,
          nchA_hbm, nchB_hbm, src_v, dst_v, bAs, bAd, bBs, bBd, cnt_v):
        c = lax.axis_index("c")
        s = lax.axis_index("s")

        @pl.when(c == 0)
        def _():
            pltpu.sync_copy(src_hbm.at[s], src_v)
            pltpu.sync_copy(dst_hbm.at[s], dst_v)
            iota = lax.iota(jnp.int32, 16)

            def row(j, off):
                offA, offB = off
                for kk in range(CH // 16):
                    sv = src_v[j, pl.ds(kk * 16, 16)]
                    dv = dst_v[j, pl.ds(kk * 16, 16)]
                    mA = dv < NH
                    mB = dv >= NH
                    cumA = plsc.cumsum(jnp.where(mA, 1, 0))
                    posA = offA + cumA - 1
                    posB = offB + iota - cumA
                    plsc.store_scatter(bAs, [posA // CH, posA % CH], sv,
                                       mask=mA)
                    plsc.store_scatter(bAd, [posA // CH, posA % CH], dv,
                                       mask=mA)
                    plsc.store_scatter(bBs, [posB // CH, posB % CH], sv,
                                       mask=mB)
                    plsc.store_scatter(bBd, [posB // CH, posB % CH],
                                       dv - NH, mask=mB)
                    nA = plsc.all_reduce_population_count(mA)
                    offA = offA + nA
                    offB = offB + 16 - nA
                return (offA, offB)

            zeros16 = jnp.zeros((16,), jnp.int32)
            offA, offB = lax.fori_loop(0, ROWS_PER_TILE, row,
                                       (zeros16, zeros16))

            zsrc = jnp.zeros((16,), jnp.int32)
            zdst = jnp.full((16,), NH, jnp.int32)
            mall = iota < 16
            for t in range(CH // 16):
                pA = offA + t * 16 + iota
                plsc.store_scatter(bAs, [pA // CH, pA % CH], zsrc, mask=mall)
                plsc.store_scatter(bAd, [pA // CH, pA % CH], zdst, mask=mall)
                pB = offB + t * 16 + iota
                plsc.store_scatter(bBs, [pB // CH, pB % CH], zsrc, mask=mall)
                plsc.store_scatter(bBd, [pB // CH, pB % CH], zdst, mask=mall)

            pltpu.sync_copy(bAs, srcA_hbm.at[s])
            pltpu.sync_copy(bAd, dstA_hbm.at[s])
            pltpu.sync_copy(bBs, srcB_hbm.at[s])
            pltpu.sync_copy(bBd, dstB_hbm.at[s])

            nchunksA = (offA + CH - 1) // CH
            cnt_v[...] = jnp.where(iota == 0, nchunksA, 0)
            pltpu.sync_copy(cnt_v, nchA_hbm.at[pl.ds(s * 16, 16)])
            nchunksB = (offB + CH - 1) // CH
            cnt_v[...] = jnp.where(iota == 0, nchunksB, 0)
            pltpu.sync_copy(cnt_v, nchB_hbm.at[pl.ds(s * 16, 16)])

    return k(src3d, dst3d)


def _my_count(nch_v, s):
    vec = nch_v[pl.ds(s * 16, 16)]
    iota = lax.iota(jnp.int32, 16)
    return jnp.sum(jnp.where(iota == 0, vec, 0))


def _deg_sc(dstA, dstB, nchA, nchB):
    @functools.partial(
        pl.kernel,
        out_type=jax.ShapeDtypeStruct((N, DH), jnp.float32),
        mesh=_mesh(),
        compiler_params=pltpu.CompilerParams(needs_layout_passes=False),
        scratch_types=[
            pltpu.VMEM((LROWS, CH), jnp.int32),
            pltpu.VMEM((16 * TILES,), jnp.int32),
            pltpu.VMEM((16 * TILES,), jnp.int32),
            pltpu.VMEM((CH, DH), jnp.float32),
            pltpu.VMEM((ZR, DH), jnp.float32),
            pltpu.VMEM_SHARED((NACC, DH), jnp.float32),
            pltpu.SemaphoreType.DMA,
        ],
    )
    def k(dstA_hbm, dstB_hbm, nchA_hbm, nchB_hbm, out_hbm,
          idx_v, nchA_v, nchB_v, ones_v, zbuf_v, acc_sh, ssem):
        c = lax.axis_index("c")
        s = lax.axis_index("s")

        _fill_vmem_rows(ones_v, CH, DH, 1.0)
        _fill_vmem_rows(zbuf_v, ZR, DH, 0.0)
        _per_tile_chunks(
            s, lambda b: pltpu.sync_copy(zbuf_v, acc_sh.at[pl.ds(b, ZR)]))
        pltpu.sync_copy(nchA_hbm, nchA_v)
        pltpu.sync_copy(nchB_hbm, nchB_v)

        @pl.when(c == 0)
        def _():
            pltpu.sync_copy(dstA_hbm.at[s], idx_v)

        @pl.when(c == 1)
        def _():
            pltpu.sync_copy(dstB_hbm.at[s], idx_v)

        n = jnp.where(c == 0, _my_count(nchA_v, s), _my_count(nchB_v, s))
        plsc.subcore_barrier()

        def body(j, carry):
            pltpu.async_copy(ones_v, acc_sh.at[idx_v.at[j]], ssem, add=True)
            return carry

        lax.fori_loop(0, n, body, 0)

        def sdrain(j, carry):
            pltpu.make_async_copy(ones_v, acc_sh.at[idx_v.at[0]], ssem).wait()
            return carry

        lax.fori_loop(0, n, sdrain, 0)
        plsc.subcore_barrier()

        def drain(b, base):
            pltpu.sync_copy(acc_sh.at[pl.ds(b, ZR)],
                            out_hbm.at[pl.ds(base + b, ZR)])

        @pl.when(c == 0)
        def _():
            _per_tile_chunks(s, lambda b: drain(b, 0))

        @pl.when(c == 1)
        def _():
            _per_tile_chunks(s, lambda b: drain(b, NH))

    return k(dstA, dstB, nchA, nchB)


def _spmm_sc(x0, x1, srcA, dstA, srcB, dstB, nchA, nchB):
    @functools.partial(
        pl.kernel,
        out_type=[jax.ShapeDtypeStruct((N, DH), jnp.float32),
                  jax.ShapeDtypeStruct((N, DH), jnp.float32)],
        mesh=_mesh(),
        compiler_params=pltpu.CompilerParams(needs_layout_passes=False),
        scratch_types=[
            pltpu.VMEM((LROWS, CH), jnp.int32),
            pltpu.VMEM((LROWS, CH), jnp.int32),
            pltpu.VMEM((16 * TILES,), jnp.int32),
            pltpu.VMEM((16 * TILES,), jnp.int32),
        ] + [pltpu.VMEM((CH, DH), jnp.float32)] * NBUF + [
            pltpu.VMEM((ZR, DH), jnp.float32),
            pltpu.VMEM_SHARED((NACC, DH), jnp.float32),
            pltpu.SemaphoreType.DMA,
            pltpu.SemaphoreType.DMA,
        ],
    )
    def k(x0_hbm, x1_hbm, srcA_hbm, dstA_hbm, srcB_hbm, dstB_hbm,
          nchA_hbm, nchB_hbm, y0_hbm, y1_hbm,
          src_v, dst_v, nchA_v, nchB_v, *rest):
        rbufs = rest[:NBUF]
        zbuf_v, acc_sh, gsem, ssem = rest[NBUF:]
        c = lax.axis_index("c")
        s = lax.axis_index("s")
        rb0 = rbufs[0]

        _fill_vmem_rows(zbuf_v, ZR, DH, 0.0)
        pltpu.sync_copy(nchA_hbm, nchA_v)
        pltpu.sync_copy(nchB_hbm, nchB_v)

        def one_pass(x_hbm, src_hbm, dst_hbm, nch_v, y_hbm, out_base):
            n = _my_count(nch_v, s)
            pltpu.sync_copy(src_hbm.at[s], src_v)
            pltpu.sync_copy(dst_hbm.at[s], dst_v)
            _per_tile_chunks(
                s, lambda b: pltpu.sync_copy(zbuf_v, acc_sh.at[pl.ds(b, ZR)]))
            plsc.subcore_barrier()

            def waitG(buf):
                pltpu.make_async_copy(x_hbm.at[src_v.at[0]], buf, gsem).wait()

            def waitS(buf):
                pltpu.make_async_copy(buf, acc_sh.at[dst_v.at[0]], ssem).wait()

            for i in range(NBUF):
                @pl.when(i < n)
                def _(i=i):
                    pltpu.async_copy(x_hbm.at[src_v.at[i]], rbufs[i], gsem)

            def body(q, carry):
                j = NBUF * q
                for i in range(NBUF):
                    ch = j + i
                    waitG(rbufs[i])
                    pltpu.async_copy(rbufs[i], acc_sh.at[dst_v.at[ch]], ssem,
                                     add=True)

                    @pl.when(ch + NBUF < n)
                    def _(i=i, ch=ch):
                        waitS(rbufs[i])
                        pltpu.async_copy(x_hbm.at[src_v.at[ch + NBUF]],
                                         rbufs[i], gsem)
                return carry

            lax.fori_loop(0, n // NBUF, body, 0)

            tail = (n // NBUF) * NBUF
            for i in range(NBUF - 1):
                @pl.when(tail + i < n)
                def _(i=i):
                    waitG(rbufs[i])
                    pltpu.async_copy(rbufs[i],
                                     acc_sh.at[dst_v.at[tail + i]], ssem,
                                     add=True)

            def sdrain(q, carry):
                waitS(rb0)
                return carry

            lax.fori_loop(0, jnp.minimum(n, NBUF), sdrain, 0)
            plsc.subcore_barrier()
            _per_tile_chunks(
                s, lambda b: pltpu.sync_copy(
                    acc_sh.at[pl.ds(b, ZR)],
                    y_hbm.at[pl.ds(out_base + b, ZR)]))

        def both_passes(x_hbm, y_hbm):
            one_pass(x_hbm, srcA_hbm, dstA_hbm, nchA_v, y_hbm, 0)
            plsc.subcore_barrier()
            one_pass(x_hbm, srcB_hbm, dstB_hbm, nchB_v, y_hbm, NH)

        @pl.when(c == 0)
        def _():
            both_passes(x0_hbm, y0_hbm)

        @pl.when(c == 1)
        def _():
            both_passes(x1_hbm, y1_hbm)

    return k(x0, x1, srcA, dstA, srcB, dstB, nchA, nchB)


def _dinv_of(deg_ref):
    return lax.rsqrt(deg_ref[:, 0:1] + 1.0)


def _half_shapes():
    return [jax.ShapeDtypeStruct((N, DH), jnp.float32) for _ in range(2)]


def _tc_xw(x, w0):
    def body(x_ref, w_ref, o_ref):
        o_ref[...] = jnp.dot(x_ref[...], w_ref[...],
                             preferred_element_type=jnp.float32)

    return pl.pallas_call(
        body,
        grid=(GRID,),
        in_specs=[
            pl.BlockSpec((RB, D), lambda i: (i, 0)),
            pl.BlockSpec((D, D), lambda i: (0, 0)),
        ],
        out_specs=pl.BlockSpec((RB, D), lambda i: (i, 0)),
        out_shape=jax.ShapeDtypeStruct((N, D), jnp.float32),
    )(x, w0)


def _tc_scale(xw, deg):
    def body(xw_ref, deg_ref, o0_ref, o1_ref, dv_ref):
        dinv = _dinv_of(deg_ref)
        xt = dinv * xw_ref[...]
        o0_ref[...] = xt[:, :DH]
        o1_ref[...] = xt[:, DH:]
        dv_ref[...] = dinv.reshape(1, 1, RB)

    half = pl.BlockSpec((RB, DH), lambda i: (i, 0))
    return pl.pallas_call(
        body,
        grid=(GRID,),
        in_specs=[
            pl.BlockSpec((RB, D), lambda i: (i, 0)),
            half,
        ],
        out_specs=[half, half, pl.BlockSpec((1, 1, RB), lambda i: (i, 0, 0))],
        out_shape=_half_shapes() + [
            jax.ShapeDtypeStruct((GRID, 1, RB), jnp.float32)],
    )(xw, deg)


def _tc_mid(y0, y1, p0, p1, dinv3, b, w):
    def body(y0_ref, y1_ref, p0_ref, p1_ref, dv_ref, b_ref, w_ref,
             o0_ref, o1_ref):
        dinv = dv_ref[...].reshape(RB, 1)
        h0 = jax.nn.relu(dinv * (y0_ref[...] + p0_ref[...]) + b_ref[0:1, :DH])
        h1 = jax.nn.relu(dinv * (y1_ref[...] + p1_ref[...]) + b_ref[0:1, DH:])
        h = jnp.concatenate([h0, h1], axis=1)
        xw = jnp.dot(h, w_ref[...], preferred_element_type=jnp.float32)
        xt = dinv * xw
        o0_ref[...] = xt[:, :DH]
        o1_ref[...] = xt[:, DH:]

    half = pl.BlockSpec((RB, DH), lambda i: (i, 0))
    return pl.pallas_call(
        body,
        grid=(GRID,),
        in_specs=[
            half, half, half, half,
            pl.BlockSpec((1, 1, RB), lambda i: (i, 0, 0)),
            pl.BlockSpec((1, D), lambda i: (0, 0)),
            pl.BlockSpec((D, D), lambda i: (0, 0)),
        ],
        out_specs=[half, half],
        out_shape=_half_shapes(),
    )(y0, y1, p0, p1, dinv3, b, w)


def _tc_final(y0, y1, p0, p1, dinv3, b2, wnT, bn, batch3, wf1, bf1, wf2,
              bf2):
    def body(y0_ref, y1_ref, p0_ref, p1_ref, dv_ref, b_ref, wn_ref, bn_ref,
             batch_ref, wf1_ref, bf1_ref, wf2_ref, bf2_ref,
             node_ref, fea_ref, pooled_acc, cnt_acc):
        i = pl.program_id(0)

        @pl.when(i == 0)
        def _():
            pooled_acc[...] = jnp.zeros_like(pooled_acc)
            cnt_acc[...] = jnp.zeros_like(cnt_acc)

        dinv = dv_ref[...].reshape(RB, 1)
        h0 = jax.nn.relu(dinv * (y0_ref[...] + p0_ref[...]) + b_ref[0:1, :DH])
        h1 = jax.nn.relu(dinv * (y1_ref[...] + p1_ref[...]) + b_ref[0:1, DH:])
        h = jnp.concatenate([h0, h1], axis=1)

        npv = jnp.sum(h * wn_ref[...], axis=1) + bn_ref[0, 0]
        node_ref[...] = jax.nn.sigmoid(npv).reshape(1, 1, RB)

        brow = batch_ref[0, 0, :]
        seg = lax.broadcasted_iota(jnp.int32, (G, RB), 0)
        onehot = jnp.where(seg == brow[None, :], 1.0, 0.0).astype(jnp.float32)
        pooled_acc[...] += jnp.dot(onehot, h,
                                   preferred_element_type=jnp.float32)
        cnt = jnp.sum(onehot, axis=1)
        cnt_acc[...] += jnp.broadcast_to(cnt[:, None], (G, 128))

        @pl.when(i == GRID - 1)
        def _():
            cntcol = cnt_acc[:, 0:1]
            pooled = pooled_acc[...] / jnp.maximum(cntcol, 1.0)
            f1 = jax.nn.relu(
                jnp.dot(pooled, wf1_ref[...],
                        preferred_element_type=jnp.float32) + bf1_ref[0:1, :])
            f2 = jnp.dot(f1, wf2_ref[...],
                         preferred_element_type=jnp.float32) + bf2_ref[0:1, :]
            fea_ref[...] = jax.nn.sigmoid(f2)

    half = pl.BlockSpec((RB, DH), lambda i: (i, 0))
    return pl.pallas_call(
        body,
        grid=(GRID,),
        in_specs=[
            half, half, half, half,
            pl.BlockSpec((1, 1, RB), lambda i: (i, 0, 0)),
            pl.BlockSpec((1, D), lambda i: (0, 0)),
            pl.BlockSpec((1, D), lambda i: (0, 0)),
            pl.BlockSpec((1, 1), lambda i: (0, 0)),
            pl.BlockSpec((1, 1, RB), lambda i: (i, 0, 0)),
            pl.BlockSpec((D, H), lambda i: (0, 0)),
            pl.BlockSpec((1, H), lambda i: (0, 0)),
            pl.BlockSpec((H, D), lambda i: (0, 0)),
            pl.BlockSpec((1, D), lambda i: (0, 0)),
        ],
        out_specs=[
            pl.BlockSpec((1, 1, RB), lambda i: (i, 0, 0)),
            pl.BlockSpec((G, D), lambda i: (0, 0)),
        ],
        out_shape=[jax.ShapeDtypeStruct((GRID, 1, RB), jnp.float32),
                   jax.ShapeDtypeStruct((G, D), jnp.float32)],
        scratch_shapes=[
            pltpu.VMEM((G, D), jnp.float32),
            pltpu.VMEM((G, 128), jnp.float32),
        ],
    )(y0, y1, p0, p1, dinv3, b2, wnT, bn, batch3, wf1, bf1, wf2, bf2)


def kernel(x, edge_index, batch, W0, b0, W1, b1, W2, b2, Wn, bn,
           Wf1, bf1, Wf2, bf2):
    src3d = edge_index[0].reshape(TILES, ROWS_PER_TILE, CH)
    dst3d = edge_index[1].reshape(TILES, ROWS_PER_TILE, CH)

    srcA, dstA, srcB, dstB, nchA, nchB = _part_sc(src3d, dst3d)
    lists = (srcA, dstA, srcB, dstB, nchA, nchB)

    deg = _deg_sc(dstA, dstB, nchA, nchB)

    xw0 = _tc_xw(x, W0)
    p0, p1, dinv3 = _tc_scale(xw0, deg)
    y0, y1 = _spmm_sc(p0, p1, *lists)
    p0, p1 = _tc_mid(y0, y1, p0, p1, dinv3, b0.reshape(1, D), W1)
    y0, y1 = _spmm_sc(p0, p1, *lists)
    p0, p1 = _tc_mid(y0, y1, p0, p1, dinv3, b1.reshape(1, D), W2)
    y0, y1 = _spmm_sc(p0, p1, *lists)

    node3, fea = _tc_final(
        y0, y1, p0, p1, dinv3, b2.reshape(1, D), Wn.reshape(1, D),
        bn.reshape(1, 1), batch.reshape(GRID, 1, RB),
        Wf1, bf1.reshape(1, H), Wf2, bf2.reshape(1, D))

    return node3.reshape(N), fea


# speedup vs baseline: 11.6093x; 1.0002x over previous
import functools

import jax
import jax.numpy as jnp
from jax import lax
from jax.experimental import pallas as pl
from jax.experimental.pallas import tpu as pltpu
from jax.experimental.pallas import tpu_sc as plsc

N = 10000
NH = N // 2
NACC = NH + 8
D = 256
DH = 128
E = 160000
G = 64
H = 512

CH = 80
NBUF = 4
TILES = 16
EPT = E // TILES
ROWS_PER_TILE = EPT // CH

NCHUNK = 125
ZR = NH // NCHUNK

RB = 400
GRID = N // RB


def _mesh():
    return plsc.VectorSubcoreMesh(core_axis_name="c", subcore_axis_name="s",
                                  num_cores=2, num_subcores=16)


def _fill_vmem_rows(ref, nrows, ncols, value):
    v = jnp.full((16,), value, jnp.float32)

    def body(i, carry):
        for k in range(ncols // 16):
            ref[i, pl.ds(k * 16, 16)] = v
        return carry

    lax.fori_loop(0, nrows, body, 0)


def _per_tile_chunks(s, fn):
    for m in range((NCHUNK + TILES - 1) // TILES):
        k = s + TILES * m

        @pl.when(k < NCHUNK)
        def _(k=k):
            fn(k * ZR)


LROWS = 128


def _part_sc(src3d, dst3d):
    list_ty = jax.ShapeDtypeStruct((TILES, LROWS, CH), jnp.int32)
    cnt_ty = jax.ShapeDtypeStruct((TILES * 16,), jnp.int32)

    @functools.partial(
        pl.kernel,
        out_type=[list_ty, list_ty, list_ty, list_ty, cnt_ty, cnt_ty],
        mesh=_mesh(),
        compiler_params=pltpu.CompilerParams(needs_layout_passes=False),
        scratch_types=[
            pltpu.VMEM((ROWS_PER_TILE, CH), jnp.int32),
            pltpu.VMEM((ROWS_PER_TILE, CH), jnp.int32),
            pltpu.VMEM((LROWS, CH), jnp.int32),
            pltpu.VMEM((LROWS, CH), jnp.int32),
            pltpu.VMEM((LROWS, CH), jnp.int32),
            pltpu.VMEM((LROWS, CH), jnp.int32),
            pltpu.VMEM((16,), jnp.int32),
        ],
    )
    def k(src_hbm, dst_hbm, srcA_hbm, dstA_hbm, srcB_hbm, dstB_hbm,
          nchA_hbm, nchB_hbm, src_v, dst_v, bAs, bAd, bBs, bBd, cnt_v):
        c = lax.axis_index("c")
        s = lax.axis_index("s")

        @pl.when(c == 0)
        def _():
            pltpu.sync_copy(src_hbm.at[s], src_v)
            pltpu.sync_copy(dst_hbm.at[s], dst_v)
            iota = lax.iota(jnp.int32, 16)

            def row(j, off):
                offA, offB = off
                for kk in range(CH // 16):
                    sv = src_v[j, pl.ds(kk * 16, 16)]
                    dv = dst_v[j, pl.ds(kk * 16, 16)]
                    mA = dv < NH
                    mB = dv >= NH
                    cumA = plsc.cumsum(jnp.where(mA, 1, 0))
                    posA = offA + cumA - 1
                    posB = offB + iota - cumA
                    plsc.store_scatter(bAs, [posA // CH, posA % CH], sv,
                                       mask=mA)
                    plsc.store_scatter(bAd, [posA // CH, posA % CH], dv,
                                       mask=mA)
                    plsc.store_scatter(bBs, [posB // CH, posB % CH], sv,
                                       mask=mB)
                    plsc.store_scatter(bBd, [posB // CH, posB % CH],
                                       dv - NH, mask=mB)
                    nA = plsc.all_reduce_population_count(mA)
                    offA = offA + nA
                    offB = offB + 16 - nA
                return (offA, offB)

            zeros16 = jnp.zeros((16,), jnp.int32)
            offA, offB = lax.fori_loop(0, ROWS_PER_TILE, row,
                                       (zeros16, zeros16))

            zsrc = jnp.zeros((16,), jnp.int32)
            zdst = jnp.full((16,), NH, jnp.int32)
            mall = iota < 16
            for t in range(CH // 16):
                pA = offA + t * 16 + iota
                plsc.store_scatter(bAs, [pA // CH, pA % CH], zsrc, mask=mall)
                plsc.store_scatter(bAd, [pA // CH, pA % CH], zdst, mask=mall)
                pB = offB + t * 16 + iota
                plsc.store_scatter(bBs, [pB // CH, pB % CH], zsrc, mask=mall)
                plsc.store_scatter(bBd, [pB // CH, pB % CH], zdst, mask=mall)

            pltpu.sync_copy(bAs, srcA_hbm.at[s])
            pltpu.sync_copy(bAd, dstA_hbm.at[s])
            pltpu.sync_copy(bBs, srcB_hbm.at[s])
            pltpu.sync_copy(bBd, dstB_hbm.at[s])

            nchunksA = (offA + CH - 1) // CH
            cnt_v[...] = jnp.where(iota == 0, nchunksA, 0)
            pltpu.sync_copy(cnt_v, nchA_hbm.at[pl.ds(s * 16, 16)])
            nchunksB = (offB + CH - 1) // CH
            cnt_v[...] = jnp.where(iota == 0, nchunksB, 0)
            pltpu.sync_copy(cnt_v, nchB_hbm.at[pl.ds(s * 16, 16)])

    return k(src3d, dst3d)


def _my_count(nch_v, s):
    vec = nch_v[pl.ds(s * 16, 16)]
    iota = lax.iota(jnp.int32, 16)
    return jnp.sum(jnp.where(iota == 0, vec, 0))


def _deg_sc(dstA, dstB, nchA, nchB):
    @functools.partial(
        pl.kernel,
        out_type=jax.ShapeDtypeStruct((N, DH), jnp.float32),
        mesh=_mesh(),
        compiler_params=pltpu.CompilerParams(needs_layout_passes=False),
        scratch_types=[
            pltpu.VMEM((LROWS, CH), jnp.int32),
            pltpu.VMEM((16 * TILES,), jnp.int32),
            pltpu.VMEM((16 * TILES,), jnp.int32),
            pltpu.VMEM((CH, DH), jnp.float32),
            pltpu.VMEM((ZR, DH), jnp.float32),
            pltpu.VMEM_SHARED((NACC, DH), jnp.float32),
            pltpu.SemaphoreType.DMA,
        ],
    )
    def k(dstA_hbm, dstB_hbm, nchA_hbm, nchB_hbm, out_hbm,
          idx_v, nchA_v, nchB_v, ones_v, zbuf_v, acc_sh, ssem):
        c = lax.axis_index("c")
        s = lax.axis_index("s")

        _fill_vmem_rows(ones_v, CH, DH, 1.0)
        _fill_vmem_rows(zbuf_v, ZR, DH, 0.0)
        _per_tile_chunks(
            s, lambda b: pltpu.sync_copy(zbuf_v, acc_sh.at[pl.ds(b, ZR)]))
        pltpu.sync_copy(nchA_hbm, nchA_v)
        pltpu.sync_copy(nchB_hbm, nchB_v)

        @pl.when(c == 0)
        def _():
            pltpu.sync_copy(dstA_hbm.at[s], idx_v)

        @pl.when(c == 1)
        def _():
            pltpu.sync_copy(dstB_hbm.at[s], idx_v)

        n = jnp.where(c == 0, _my_count(nchA_v, s), _my_count(nchB_v, s))
        plsc.subcore_barrier()

        def body(j, carry):
            pltpu.async_copy(ones_v, acc_sh.at[idx_v.at[j]], ssem, add=True)
            return carry

        lax.fori_loop(0, n, body, 0)

        def sdrain(j, carry):
            pltpu.make_async_copy(ones_v, acc_sh.at[idx_v.at[0]], ssem).wait()
            return carry

        lax.fori_loop(0, n, sdrain, 0)
        plsc.subcore_barrier()

        def drain(b, base):
            pltpu.sync_copy(acc_sh.at[pl.ds(b, ZR)],
                            out_hbm.at[pl.ds(base + b, ZR)])

        @pl.when(c == 0)
        def _():
            _per_tile_chunks(s, lambda b: drain(b, 0))

        @pl.when(c == 1)
        def _():
            _per_tile_chunks(s, lambda b: drain(b, NH))

    return k(dstA, dstB, nchA, nchB)


def _spmm_sc(x0, x1, srcA, dstA, srcB, dstB, nchA, nchB):
    @functools.partial(
        pl.kernel,
        out_type=[jax.ShapeDtypeStruct((N, DH), jnp.float32),
                  jax.ShapeDtypeStruct((N, DH), jnp.float32)],
        mesh=_mesh(),
        compiler_params=pltpu.CompilerParams(needs_layout_passes=False),
        scratch_types=[
            pltpu.VMEM((LROWS, CH), jnp.int32),
            pltpu.VMEM((LROWS, CH), jnp.int32),
            pltpu.VMEM((16 * TILES,), jnp.int32),
            pltpu.VMEM((16 * TILES,), jnp.int32),
        ] + [pltpu.VMEM((CH, DH), jnp.float32)] * NBUF + [
            pltpu.VMEM((ZR, DH), jnp.float32),
            pltpu.VMEM_SHARED((NACC, DH), jnp.float32),
            pltpu.SemaphoreType.DMA,
            pltpu.SemaphoreType.DMA,
        ],
    )
    def k(x0_hbm, x1_hbm, srcA_hbm, dstA_hbm, srcB_hbm, dstB_hbm,
          nchA_hbm, nchB_hbm, y0_hbm, y1_hbm,
          src_v, dst_v, nchA_v, nchB_v, *rest):
        rbufs = rest[:NBUF]
        zbuf_v, acc_sh, gsem, ssem = rest[NBUF:]
        c = lax.axis_index("c")
        s = lax.axis_index("s")
        rb0 = rbufs[0]

        _fill_vmem_rows(zbuf_v, ZR, DH, 0.0)
        pltpu.sync_copy(nchA_hbm, nchA_v)
        pltpu.sync_copy(nchB_hbm, nchB_v)

        def one_pass(x_hbm, src_hbm, dst_hbm, nch_v, y_hbm, out_base):
            n = _my_count(nch_v, s)
            pltpu.sync_copy(src_hbm.at[s], src_v)
            pltpu.sync_copy(dst_hbm.at[s], dst_v)
            _per_tile_chunks(
                s, lambda b: pltpu.sync_copy(zbuf_v, acc_sh.at[pl.ds(b, ZR)]))
            plsc.subcore_barrier()

            def waitG(buf):
                pltpu.make_async_copy(x_hbm.at[src_v.at[0]], buf, gsem).wait()

            def waitS(buf):
                pltpu.make_async_copy(buf, acc_sh.at[dst_v.at[0]], ssem).wait()

            for i in range(NBUF):
                @pl.when(i < n)
                def _(i=i):
                    pltpu.async_copy(x_hbm.at[src_v.at[i]], rbufs[i], gsem)

            def body(q, carry):
                j = NBUF * q
                for i in range(NBUF):
                    ch = j + i
                    waitG(rbufs[i])
                    pltpu.async_copy(rbufs[i], acc_sh.at[dst_v.at[ch]], ssem,
                                     add=True)

                    @pl.when(ch + NBUF < n)
                    def _(i=i, ch=ch):
                        waitS(rbufs[i])
                        pltpu.async_copy(x_hbm.at[src_v.at[ch + NBUF]],
                                         rbufs[i], gsem)
                return carry

            lax.fori_loop(0, n // NBUF, body, 0)

            tail = (n // NBUF) * NBUF
            for i in range(NBUF - 1):
                @pl.when(tail + i < n)
                def _(i=i):
                    waitG(rbufs[i])
                    pltpu.async_copy(rbufs[i],
                                     acc_sh.at[dst_v.at[tail + i]], ssem,
                                     add=True)

            def sdrain(q, carry):
                waitS(rb0)
                return carry

            lax.fori_loop(0, jnp.minimum(n, NBUF), sdrain, 0)
            plsc.subcore_barrier()
            _per_tile_chunks(
                s, lambda b: pltpu.sync_copy(
                    acc_sh.at[pl.ds(b, ZR)],
                    y_hbm.at[pl.ds(out_base + b, ZR)]))

        def both_passes(x_hbm, y_hbm):
            one_pass(x_hbm, srcA_hbm, dstA_hbm, nchA_v, y_hbm, 0)
            plsc.subcore_barrier()
            one_pass(x_hbm, srcB_hbm, dstB_hbm, nchB_v, y_hbm, NH)

        @pl.when(c == 0)
        def _():
            both_passes(x0_hbm, y0_hbm)

        @pl.when(c == 1)
        def _():
            both_passes(x1_hbm, y1_hbm)

    return k(x0, x1, srcA, dstA, srcB, dstB, nchA, nchB)


def _dinv_of(deg_ref):
    return lax.rsqrt(deg_ref[:, 0:1] + 1.0)


def _half_shapes():
    return [jax.ShapeDtypeStruct((N, DH), jnp.float32) for _ in range(2)]


def _tc_xw(x, w0):
    def body(x_ref, w_ref, o_ref):
        o_ref[...] = jnp.dot(x_ref[...], w_ref[...],
                             preferred_element_type=jnp.float32)

    return pl.pallas_call(
        body,
        grid=(GRID,),
        in_specs=[
            pl.BlockSpec((RB, D), lambda i: (i, 0)),
            pl.BlockSpec((D, D), lambda i: (0, 0)),
        ],
        out_specs=pl.BlockSpec((RB, D), lambda i: (i, 0)),
        out_shape=jax.ShapeDtypeStruct((N, D), jnp.float32),
    )(x, w0)


def _tc_scale(xw, deg):
    def body(xw_ref, deg_ref, o0_ref, o1_ref, dv_ref):
        dinv = _dinv_of(deg_ref)
        xt = dinv * xw_ref[...]
        o0_ref[...] = xt[:, :DH]
        o1_ref[...] = xt[:, DH:]
        dv_ref[...] = dinv.reshape(1, 1, RB)

    half = pl.BlockSpec((RB, DH), lambda i: (i, 0))
    return pl.pallas_call(
        body,
        grid=(GRID,),
        in_specs=[
            pl.BlockSpec((RB, D), lambda i: (i, 0)),
            half,
        ],
        out_specs=[half, half, pl.BlockSpec((1, 1, RB), lambda i: (i, 0, 0))],
        out_shape=_half_shapes() + [
            jax.ShapeDtypeStruct((GRID, 1, RB), jnp.float32)],
    )(xw, deg)


def _tc_mid(y0, y1, p0, p1, dinv3, b, w):
    def body(y0_ref, y1_ref, p0_ref, p1_ref, dv_ref, b_ref, w_ref,
             o0_ref, o1_ref):
        dinv = dv_ref[...].reshape(RB, 1)
        h0 = jax.nn.relu(dinv * (y0_ref[...] + p0_ref[...]) + b_ref[0:1, :DH])
        h1 = jax.nn.relu(dinv * (y1_ref[...] + p1_ref[...]) + b_ref[0:1, DH:])
        h = jnp.concatenate([h0, h1], axis=1)
        xw = jnp.dot(h, w_ref[...], preferred_element_type=jnp.float32)
        xt = dinv * xw
        o0_ref[...] = xt[:, :DH]
        o1_ref[...] = xt[:, DH:]

    half = pl.BlockSpec((RB, DH), lambda i: (i, 0))
    return pl.pallas_call(
        body,
        grid=(GRID,),
        in_specs=[
            half, half, half, half,
            pl.BlockSpec((1, 1, RB), lambda i: (i, 0, 0)),
            pl.BlockSpec((1, D), lambda i: (0, 0)),
            pl.BlockSpec((D, D), lambda i: (0, 0)),
        ],
        out_specs=[half, half],
        out_shape=_half_shapes(),
    )(y0, y1, p0, p1, dinv3, b, w)


def _tc_final(y0, y1, p0, p1, dinv3, b2, wnT, bn, batch3, wf1, bf1, wf2,
              bf2):
    def body(y0_ref, y1_ref, p0_ref, p1_ref, dv_ref, b_ref, wn_ref, bn_ref,
             batch_ref, wf1_ref, bf1_ref, wf2_ref, bf2_ref,
             node_ref, fea_ref, pooled_acc, cnt_acc):
        i = pl.program_id(0)

        @pl.when(i == 0)
        def _():
            pooled_acc[...] = jnp.zeros_like(pooled_acc)
            cnt_acc[...] = jnp.zeros_like(cnt_acc)

        dinv = dv_ref[...].reshape(RB, 1)
        h0 = jax.nn.relu(dinv * (y0_ref[...] + p0_ref[...]) + b_ref[0:1, :DH])
        h1 = jax.nn.relu(dinv * (y1_ref[...] + p1_ref[...]) + b_ref[0:1, DH:])
        h = jnp.concatenate([h0, h1], axis=1)

        npv = jnp.sum(h * wn_ref[...], axis=1) + bn_ref[0, 0]
        node_ref[...] = jax.nn.sigmoid(npv).reshape(1, 1, RB)

        brow = batch_ref[0, 0, :]
        seg = lax.broadcasted_iota(jnp.int32, (G, RB), 0)
        onehot = jnp.where(seg == brow[None, :], 1.0, 0.0).astype(jnp.float32)
        pooled_acc[...] += jnp.dot(onehot, h,
                                   preferred_element_type=jnp.float32)
        cnt = jnp.sum(onehot, axis=1)
        cnt_acc[...] += jnp.broadcast_to(cnt[:, None], (G, 128))

        @pl.when(i == GRID - 1)
        def _():
            cntcol = cnt_acc[:, 0:1]
            pooled = pooled_acc[...] / jnp.maximum(cntcol, 1.0)
            f1 = jax.nn.relu(
                jnp.dot(pooled, wf1_ref[...],
                        preferred_element_type=jnp.float32) + bf1_ref[0:1, :])
            f2 = jnp.dot(f1, wf2_ref[...],
                         preferred_element_type=jnp.float32) + bf2_ref[0:1, :]
            fea_ref[...] = jax.nn.sigmoid(f2)

    half = pl.BlockSpec((RB, DH), lambda i: (i, 0))
    return pl.pallas_call(
        body,
        grid=(GRID,),
        in_specs=[
            half, half, half, half,
            pl.BlockSpec((1, 1, RB), lambda i: (i, 0, 0)),
            pl.BlockSpec((1, D), lambda i: (0, 0)),
            pl.BlockSpec((1, D), lambda i: (0, 0)),
            pl.BlockSpec((1, 1), lambda i: (0, 0)),
            pl.BlockSpec((1, 1, RB), lambda i: (i, 0, 0)),
            pl.BlockSpec((D, H), lambda i: (0, 0)),
            pl.BlockSpec((1, H), lambda i: (0, 0)),
            pl.BlockSpec((H, D), lambda i: (0, 0)),
            pl.BlockSpec((1, D), lambda i: (0, 0)),
        ],
        out_specs=[
            pl.BlockSpec((1, 1, RB), lambda i: (i, 0, 0)),
            pl.BlockSpec((G, D), lambda i: (0, 0)),
        ],
        out_shape=[jax.ShapeDtypeStruct((GRID, 1, RB), jnp.float32),
                   jax.ShapeDtypeStruct((G, D), jnp.float32)],
        scratch_shapes=[
            pltpu.VMEM((G, D), jnp.float32),
            pltpu.VMEM((G, 128), jnp.float32),
        ],
    )(y0, y1, p0, p1, dinv3, b2, wnT, bn, batch3, wf1, bf1, wf2, bf2)


def kernel(x, edge_index, batch, W0, b0, W1, b1, W2, b2, Wn, bn,
           Wf1, bf1, Wf2, bf2):
    src3d = edge_index[0].reshape(TILES, ROWS_PER_TILE, CH)
    dst3d = edge_index[1].reshape(TILES, ROWS_PER_TILE, CH)

    srcA, dstA, srcB, dstB, nchA, nchB = _part_sc(src3d, dst3d)
    lists = (srcA, dstA, srcB, dstB, nchA, nchB)

    deg = _deg_sc(dstA, dstB, nchA, nchB)

    xw0 = _tc_xw(x, W0)
    p0, p1, dinv3 = _tc_scale(xw0, deg)
    y0, y1 = _spmm_sc(p0, p1, *lists)
    p0, p1 = _tc_mid(y0, y1, p0, p1, dinv3, b0.reshape(1, D), W1)
    y0, y1 = _spmm_sc(p0, p1, *lists)
    p0, p1 = _tc_mid(y0, y1, p0, p1, dinv3, b1.reshape(1, D), W2)
    y0, y1 = _spmm_sc(p0, p1, *lists)

    node3, fea = _tc_final(
        y0, y1, p0, p1, dinv3, b2.reshape(1, D), Wn.reshape(1, D),
        bn.reshape(1, 1), batch.reshape(GRID, 1, RB),
        Wf1, bf1.reshape(1, H), Wf2, bf2.reshape(1, D))

    return node3.reshape(N), fea
